# Initial kernel scaffold; baseline (speedup 1.0000x reference)
import jax
import jax.numpy as jnp
from jax.experimental import pallas as pl


def kernel(x, enc_in_w, enc_in_b, enc_l0_rb0_conv1_w, enc_l0_rb0_conv1_b, enc_l0_rb0_bn1_g, enc_l0_rb0_bn1_b, enc_l0_rb0_conv2_w, enc_l0_rb0_conv2_b, enc_l0_rb0_bn2_g, enc_l0_rb0_bn2_b, enc_l0_down_w, enc_l0_down_b, enc_l1_rb0_conv1_w, enc_l1_rb0_conv1_b, enc_l1_rb0_bn1_g, enc_l1_rb0_bn1_b, enc_l1_rb0_conv2_w, enc_l1_rb0_conv2_b, enc_l1_rb0_bn2_g, enc_l1_rb0_bn2_b, enc_l1_down_w, enc_l1_down_b, enc_out_w, enc_out_b, dec_in_w, dec_in_b, dec_l0_rb0_conv1_w, dec_l0_rb0_conv1_b, dec_l0_rb0_bn1_g, dec_l0_rb0_bn1_b, dec_l0_rb0_conv2_w, dec_l0_rb0_conv2_b, dec_l0_rb0_bn2_g, dec_l0_rb0_bn2_b, dec_l0_up_w, dec_l0_up_b, dec_l1_rb0_conv1_w, dec_l1_rb0_conv1_b, dec_l1_rb0_bn1_g, dec_l1_rb0_bn1_b, dec_l1_rb0_conv2_w, dec_l1_rb0_conv2_b, dec_l1_rb0_bn2_g, dec_l1_rb0_bn2_b, dec_l1_up_w, dec_l1_up_b, dec_out_w, dec_out_b, codebook):
    raise NotImplementedError("write your pallas kernel here")


# speedup vs baseline: 14.5900x; 14.5900x over previous
import functools

import jax
import jax.numpy as jnp
from jax import lax
from jax.experimental import pallas as pl
from jax.experimental.pallas import tpu as pltpu

C = 128
TM = 256
VMEM_LIMIT = 32 * 1024 * 1024
_F32 = jnp.float32
_BF16 = jnp.bfloat16


def _w_taps(w):
    _, _, kh, kw = w.shape
    wt = jnp.transpose(w, (2, 3, 1, 0))
    return wt.reshape(kh * kw * w.shape[1], w.shape[0]).astype(_BF16)


def _w_convt(w):
    sel = ((3, 1), (2, 0))
    zero = jnp.zeros_like(w[:, :, 0, 0])
    taps = []
    for r in range(3):
        for c in range(3):
            blocks = []
            for a in (0, 1):
                for b in (0, 1):
                    di, dj = r - a, c - b
                    if 0 <= di <= 1 and 0 <= dj <= 1:
                        blocks.append(w[:, :, sel[a][di], sel[b][dj]])
                    else:
                        blocks.append(zero)
            taps.append(jnp.concatenate(blocks, axis=1))
    return jnp.concatenate(taps, axis=0).astype(_BF16)


def _bn_scale_shift(s, q, count, gamma, beta, eps=1e-5):
    mean = s / count
    var = jnp.maximum(q / count - mean * mean, 0.0)
    scale = gamma * lax.rsqrt(var + eps)
    shift = beta - mean * scale
    return scale.reshape(1, C).astype(_F32), shift.reshape(1, C).astype(_F32)


def _halo(a):
    return jnp.pad(a, ((0, 0), (1, 1), (1, 1), (0, 0)))


def _slab_dma(xp, kh, kw, ho, wo, nimg, slab_ref, slab2_ref, sem):
    rows = nimg * ho * wo
    for i in range(kh):
        for j in range(kw):
            t = i * kw + j
            slab_ref[:, t * C:(t + 1) * C] = (
                xp[:, i:i + ho, j:j + wo, :].reshape(rows, C))
    cp = pltpu.make_async_copy(slab_ref, slab2_ref, sem)
    cp.start()
    cp.wait()


def _gemm_chunks(slab2_ref, w_ref, b_ref, rows):
    n = w_ref.shape[-1]
    tn = 256 if (n % 256 == 0 and n >= 256) else n
    for r in range(rows // TM):
        a_c = slab2_ref[r * TM:(r + 1) * TM, :]
        if tn == n:
            yc = jnp.dot(a_c, w_ref[...], preferred_element_type=_F32) + b_ref[...]
        else:
            yc = jnp.concatenate(
                [jnp.dot(a_c, w_ref[:, c * tn:(c + 1) * tn],
                         preferred_element_type=_F32)
                 for c in range(n // tn)], axis=1) + b_ref[...]
        yield r, yc


def _in_conv1_body(x_ref, wi_ref, bi_ref, w_ref, b_ref,
                   y0_ref, y1_ref, s_ref, q_ref, slab_ref, slab2_ref, sem,
                   *, nimg, h, w):
    rows = nimg * h * w
    cin = x_ref.shape[-1]
    y0 = jnp.dot(x_ref[...].reshape(rows, cin), wi_ref[...],
                 preferred_element_type=_F32) + bi_ref[...]
    y0 = y0.astype(_BF16)
    y0_ref[...] = y0
    _slab_dma(_halo(y0.reshape(nimg, h, w, C)), 3, 3, h, w, nimg,
              slab_ref, slab2_ref, sem)
    for r, yc in _gemm_chunks(slab2_ref, w_ref, b_ref, rows):
        s_ref[r:r + 1, :, :] = jnp.sum(yc, axis=0, keepdims=True)[None]
        q_ref[r:r + 1, :, :] = jnp.sum(yc * yc, axis=0, keepdims=True)[None]
        y1_ref[r * TM:(r + 1) * TM, :] = yc.astype(_BF16)


def _conv1_body(x_ref, w_ref, b_ref, o_ref, s_ref, q_ref,
                slab_ref, slab2_ref, sem, *, nimg, h, w):
    rows = nimg * h * w
    _slab_dma(_halo(x_ref[...]), 3, 3, h, w, nimg, slab_ref, slab2_ref, sem)
    for r, yc in _gemm_chunks(slab2_ref, w_ref, b_ref, rows):
        s_ref[r:r + 1, :, :] = jnp.sum(yc, axis=0, keepdims=True)[None]
        q_ref[r:r + 1, :, :] = jnp.sum(yc * yc, axis=0, keepdims=True)[None]
        o_ref[r * TM:(r + 1) * TM, :] = yc.astype(_BF16)


def _aff_conv2_body(x_ref, sc_ref, sh_ref, w_ref, b_ref,
                    o_ref, s_ref, q_ref, slab_ref, slab2_ref, sem,
                    *, nimg, h, w):
    rows = nimg * h * w
    a = jnp.maximum(x_ref[...].astype(_F32) * sc_ref[...] + sh_ref[...], 0.0)
    a = a.astype(_BF16)
    _slab_dma(_halo(a), 3, 3, h, w, nimg, slab_ref, slab2_ref, sem)
    for r, yc in _gemm_chunks(slab2_ref, w_ref, b_ref, rows):
        s_ref[r:r + 1, :, :] = jnp.sum(yc, axis=0, keepdims=True)[None]
        q_ref[r:r + 1, :, :] = jnp.sum(yc * yc, axis=0, keepdims=True)[None]
        o_ref[r * TM:(r + 1) * TM, :] = yc.astype(_BF16)


def _tail_down_body(x0_ref, x1_ref, r0_ref, r1_ref, sc_ref, sh_ref,
                    w_ref, b_ref, o_ref, slab_ref, slab2_ref, sem,
                    *, nimg, h, w):
    ho, wo = h // 2, w // 2
    rows = nimg * ho * wo
    app = {}
    for a, xr, rr in ((0, x0_ref, r0_ref), (1, x1_ref, r1_ref)):
        xe = xr[...].reshape(nimg, ho, wo, 2 * C).astype(_F32)
        re = rr[...].reshape(nimg, ho, wo, 2 * C).astype(_F32)
        for b in (0, 1):
            act = jnp.maximum(
                xe[..., b * C:(b + 1) * C] * sc_ref[...] + sh_ref[...]
                + re[..., b * C:(b + 1) * C], 0.0).astype(_BF16)
            app[(a, b)] = jnp.pad(act, ((0, 0), (a, 1 - a), (b, 1 - b), (0, 0)))
    for i in range(4):
        for j in range(4):
            t = i * 4 + j
            p = app[(1 - i % 2, 1 - j % 2)]
            slab_ref[:, t * C:(t + 1) * C] = (
                p[:, i // 2:i // 2 + ho, j // 2:j // 2 + wo, :].reshape(rows, C))
    cp = pltpu.make_async_copy(slab_ref, slab2_ref, sem)
    cp.start()
    cp.wait()
    for r, yc in _gemm_chunks(slab2_ref, w_ref, b_ref, rows):
        o_ref[r * TM:(r + 1) * TM, :] = jnp.maximum(yc, 0.0).astype(_BF16)


def _tail_convt_body(x_ref, r_ref, sc_ref, sh_ref, w_ref, b_ref, o_ref,
                     slab_ref, slab2_ref, sem, *, nimg, h, w):
    rows = nimg * h * w
    a = jnp.maximum(x_ref[...].astype(_F32) * sc_ref[...] + sh_ref[...]
                    + r_ref[...].astype(_F32), 0.0).astype(_BF16)
    _slab_dma(_halo(a), 3, 3, h, w, nimg, slab_ref, slab2_ref, sem)
    for r, yc in _gemm_chunks(slab2_ref, w_ref, b_ref, rows):
        o_ref[r * TM:(r + 1) * TM, :] = jnp.maximum(yc, 0.0).astype(_BF16)


def _tail_convt_out_body(x_ref, r_ref, sc_ref, sh_ref, w_ref, b_ref,
                         wo_ref, bo_ref, p0_ref, p1_ref, p2_ref, p3_ref,
                         slab_ref, slab2_ref, sem, *, nimg, h, w):
    rows = nimg * h * w
    a = jnp.maximum(x_ref[...].astype(_F32) * sc_ref[...] + sh_ref[...]
                    + r_ref[...].astype(_F32), 0.0).astype(_BF16)
    _slab_dma(_halo(a), 3, 3, h, w, nimg, slab_ref, slab2_ref, sem)
    outs = (p0_ref, p1_ref, p2_ref, p3_ref)
    for r, yc in _gemm_chunks(slab2_ref, w_ref, b_ref, rows):
        y4 = jnp.maximum(yc, 0.0).astype(_BF16)
        for p, o_ref in enumerate(outs):
            yp = jnp.dot(y4[:, p * C:(p + 1) * C], wo_ref[...],
                         preferred_element_type=_F32) + bo_ref[...]
            o_ref[r * TM:(r + 1) * TM, :] = jax.nn.sigmoid(yp)


def _bridge_body(y_ref, wo_ref, bo_ref, e_ref, e2_ref, wd_ref, bd_ref,
                 idx_ref, cnt_ref, h_ref):
    rows = y_ref.shape[0]
    kdim = e_ref.shape[0]
    cnt = jnp.zeros((1, kdim), _F32)
    for r in range(rows // TM):
        sl = slice(r * TM, (r + 1) * TM)
        z = jnp.dot(y_ref[sl, :], wo_ref[...],
                    preferred_element_type=_F32) + bo_ref[...]
        z2 = jnp.sum(z * z, axis=-1, keepdims=True)
        cross = lax.dot_general(z, e_ref[...], (((1,), (1,)), ((), ())),
                                preferred_element_type=_F32)
        d = z2 - 2.0 * cross + e2_ref[...]
        d_min = jnp.min(d, axis=-1, keepdims=True)
        ids = lax.broadcasted_iota(jnp.int32, d.shape, 1)
        idx = jnp.min(jnp.where(d <= d_min, ids, kdim), axis=-1, keepdims=True)
        idx_ref[sl, :] = idx
        cnt = cnt + jnp.sum((ids == idx).astype(_F32), axis=0, keepdims=True)
        hd = jnp.dot(z.astype(_BF16), wd_ref[...],
                     preferred_element_type=_F32) + bd_ref[...]
        h_ref[sl, :] = hd.astype(_BF16)
    cnt_ref[...] = cnt[None]


def _pcall(body, grid, in_specs, out_specs, out_shape, args, slab=None):
    scratch = []
    if slab is not None:
        scratch = [pltpu.VMEM(slab, _BF16), pltpu.VMEM(slab, _BF16),
                   pltpu.SemaphoreType.DMA]
    return pl.pallas_call(
        body,
        out_shape=out_shape,
        grid_spec=pltpu.PrefetchScalarGridSpec(
            num_scalar_prefetch=0, grid=grid,
            in_specs=in_specs, out_specs=out_specs,
            scratch_shapes=scratch),
        compiler_params=pltpu.CompilerParams(
            dimension_semantics=("parallel",),
            vmem_limit_bytes=VMEM_LIMIT),
    )(*args)


def _img_spec(nimg, h, w, ch):
    return pl.BlockSpec((nimg, h, w, ch), lambda i: (i, 0, 0, 0))


def _row_spec(rows, ch):
    return pl.BlockSpec((rows, ch), lambda i: (i, 0))


def _fix_spec(shape):
    nd = len(shape)
    return pl.BlockSpec(shape, lambda i: (0,) * nd)


def _stat_specs_shapes(b, nimg, h, w):
    nchunk = b * h * w // TM
    per = nimg * h * w // TM
    spec = pl.BlockSpec((per, 1, C), lambda i: (i, 0, 0))
    shape = jax.ShapeDtypeStruct((nchunk, 1, C), _F32)
    return (spec, spec), (shape, shape)


def _conv_block(x, w9, bias, *, nimg, aff=None, fuse_in=None):
    b, h, w, _ = x.shape
    rows = nimg * h * w
    grid = (b // nimg,)
    o_shape = jax.ShapeDtypeStruct((b * h * w, C), _BF16)
    st_specs, st_shapes = _stat_specs_shapes(b, nimg, h, w)
    bias = bias.reshape(1, -1).astype(_F32)
    if fuse_in is not None:
        wi, bi = fuse_in
        body = functools.partial(_in_conv1_body, nimg=nimg, h=h, w=w)
        in_specs = [_img_spec(nimg, h, w, x.shape[-1]), _fix_spec(wi.shape),
                    _fix_spec((1, C)), _fix_spec(w9.shape), _fix_spec((1, C))]
        out_specs = (_row_spec(rows, C), _row_spec(rows, C)) + st_specs
        out_shape = (o_shape, o_shape) + st_shapes
        args = (x, wi, bi.reshape(1, C).astype(_F32), w9, bias)
    elif aff is not None:
        sc, sh = aff
        body = functools.partial(_aff_conv2_body, nimg=nimg, h=h, w=w)
        in_specs = [_img_spec(nimg, h, w, C), _fix_spec((1, C)), _fix_spec((1, C)),
                    _fix_spec(w9.shape), _fix_spec((1, C))]
        out_specs = (_row_spec(rows, C),) + st_specs
        out_shape = (o_shape,) + st_shapes
        args = (x, sc, sh, w9, bias)
    else:
        body = functools.partial(_conv1_body, nimg=nimg, h=h, w=w)
        in_specs = [_img_spec(nimg, h, w, C), _fix_spec(w9.shape), _fix_spec((1, C))]
        out_specs = (_row_spec(rows, C),) + st_specs
        out_shape = (o_shape,) + st_shapes
        args = (x, w9, bias)
    res = _pcall(body, grid, in_specs, out_specs, out_shape, args,
                 slab=(rows, 9 * C))
    res = (res[0].reshape(b, h, w, C),) + tuple(res[1:])
    if fuse_in is not None:
        res = (res[0], res[1].reshape(b, h, w, C)) + tuple(res[2:])
    return res


def _down_block(y2, res, sc, sh, wd, bd, *, nimg):
    b, h, w, _ = y2.shape
    ho, wo = h // 2, w // 2
    rows = nimg * ho * wo
    grid = (b // nimg,)
    body = functools.partial(_tail_down_body, nimg=nimg, h=h, w=w)
    y2v = y2.reshape(b, ho, 2, wo, 2 * C)
    resv = res.reshape(b, ho, 2, wo, 2 * C)

    def _phase_spec(e):
        return pl.BlockSpec((nimg, ho, 1, wo, 2 * C),
                            lambda i, e=e: (i, 0, e, 0, 0))

    in_specs = [_phase_spec(0), _phase_spec(1), _phase_spec(0), _phase_spec(1),
                _fix_spec((1, C)), _fix_spec((1, C)),
                _fix_spec(wd.shape), _fix_spec((1, C))]
    out = _pcall(body, grid, in_specs, _row_spec(rows, C),
                 jax.ShapeDtypeStruct((b * ho * wo, C), _BF16),
                 (y2v, y2v, resv, resv, sc, sh, wd,
                  bd.reshape(1, C).astype(_F32)),
                 slab=(rows, 16 * C))
    return out.reshape(b, ho, wo, C)


def _convt_block(y2, res, sc, sh, wu, bu4, *, nimg):
    b, h, w, _ = y2.shape
    rows = nimg * h * w
    grid = (b // nimg,)
    body = functools.partial(_tail_convt_body, nimg=nimg, h=h, w=w)
    in_specs = [_img_spec(nimg, h, w, C), _img_spec(nimg, h, w, C),
                _fix_spec((1, C)), _fix_spec((1, C)),
                _fix_spec(wu.shape), _fix_spec((1, 4 * C))]
    y4 = _pcall(body, grid, in_specs, _row_spec(rows, 4 * C),
                jax.ShapeDtypeStruct((b * h * w, 4 * C), _BF16),
                (y2, res, sc, sh, wu, bu4), slab=(rows, 9 * C))
    y4 = y4.reshape(b, h, w, 2, 2, C)
    return jnp.transpose(y4, (0, 1, 3, 2, 4, 5)).reshape(b, 2 * h, 2 * w, C)


def _convt_out_block(y2, res, sc, sh, wu, bu4, wo, bo, *, nimg):
    b, h, w, _ = y2.shape
    rows = nimg * h * w
    grid = (b // nimg,)
    couts = wo.shape[-1]
    body = functools.partial(_tail_convt_out_body, nimg=nimg, h=h, w=w)
    in_specs = [_img_spec(nimg, h, w, C), _img_spec(nimg, h, w, C),
                _fix_spec((1, C)), _fix_spec((1, C)),
                _fix_spec(wu.shape), _fix_spec((1, 4 * C)),
                _fix_spec(wo.shape), _fix_spec((1, couts))]
    p_spec = _row_spec(rows, couts)
    p_shape = jax.ShapeDtypeStruct((b * h * w, couts), _F32)
    ps = _pcall(body, grid, in_specs, (p_spec,) * 4, (p_shape,) * 4,
                (y2, res, sc, sh, wu, bu4, wo, bo), slab=(rows, 9 * C))
    return tuple(p.reshape(b, h, w, couts) for p in ps)


def kernel(x, enc_in_w, enc_in_b,
           enc_l0_rb0_conv1_w, enc_l0_rb0_conv1_b, enc_l0_rb0_bn1_g, enc_l0_rb0_bn1_b,
           enc_l0_rb0_conv2_w, enc_l0_rb0_conv2_b, enc_l0_rb0_bn2_g, enc_l0_rb0_bn2_b,
           enc_l0_down_w, enc_l0_down_b,
           enc_l1_rb0_conv1_w, enc_l1_rb0_conv1_b, enc_l1_rb0_bn1_g, enc_l1_rb0_bn1_b,
           enc_l1_rb0_conv2_w, enc_l1_rb0_conv2_b, enc_l1_rb0_bn2_g, enc_l1_rb0_bn2_b,
           enc_l1_down_w, enc_l1_down_b,
           enc_out_w, enc_out_b,
           dec_in_w, dec_in_b,
           dec_l0_rb0_conv1_w, dec_l0_rb0_conv1_b, dec_l0_rb0_bn1_g, dec_l0_rb0_bn1_b,
           dec_l0_rb0_conv2_w, dec_l0_rb0_conv2_b, dec_l0_rb0_bn2_g, dec_l0_rb0_bn2_b,
           dec_l0_up_w, dec_l0_up_b,
           dec_l1_rb0_conv1_w, dec_l1_rb0_conv1_b, dec_l1_rb0_bn1_g, dec_l1_rb0_bn1_b,
           dec_l1_rb0_conv2_w, dec_l1_rb0_conv2_b, dec_l1_rb0_bn2_g, dec_l1_rb0_bn2_b,
           dec_l1_up_w, dec_l1_up_b,
           dec_out_w, dec_out_b,
           codebook):
    b = x.shape[0]
    num_emb, emb_dim = codebook.shape

    x8 = jnp.transpose(x, (0, 2, 3, 1)).astype(_BF16)
    cin8 = 8
    x8 = jnp.pad(x8, ((0, 0), (0, 0), (0, 0), (0, cin8 - x8.shape[-1])))
    w_in = jnp.pad(jnp.transpose(enc_in_w[:, :, 0, 0]),
                   ((0, cin8 - enc_in_w.shape[1]), (0, 0))).astype(_BF16)

    m64 = b * 64 * 64
    y0, y1, s1, q1 = _conv_block(x8, _w_taps(enc_l0_rb0_conv1_w),
                                 enc_l0_rb0_conv1_b, nimg=1,
                                 fuse_in=(w_in, enc_in_b))
    sc, sh = _bn_scale_shift(jnp.sum(s1, axis=(0, 1)), jnp.sum(q1, axis=(0, 1)),
                             m64, enc_l0_rb0_bn1_g, enc_l0_rb0_bn1_b)
    y2, s2, q2 = _conv_block(y1, _w_taps(enc_l0_rb0_conv2_w),
                             enc_l0_rb0_conv2_b, nimg=1, aff=(sc, sh))
    sc, sh = _bn_scale_shift(jnp.sum(s2, axis=(0, 1)), jnp.sum(q2, axis=(0, 1)),
                             m64, enc_l0_rb0_bn2_g, enc_l0_rb0_bn2_b)
    d0 = _down_block(y2, y0, sc, sh, _w_taps(enc_l0_down_w), enc_l0_down_b,
                     nimg=1)

    m32 = b * 32 * 32
    y1, s1, q1 = _conv_block(d0, _w_taps(enc_l1_rb0_conv1_w),
                             enc_l1_rb0_conv1_b, nimg=4)
    sc, sh = _bn_scale_shift(jnp.sum(s1, axis=(0, 1)), jnp.sum(q1, axis=(0, 1)),
                             m32, enc_l1_rb0_bn1_g, enc_l1_rb0_bn1_b)
    y2, s2, q2 = _conv_block(y1, _w_taps(enc_l1_rb0_conv2_w),
                             enc_l1_rb0_conv2_b, nimg=4, aff=(sc, sh))
    sc, sh = _bn_scale_shift(jnp.sum(s2, axis=(0, 1)), jnp.sum(q2, axis=(0, 1)),
                             m32, enc_l1_rb0_bn2_g, enc_l1_rb0_bn2_b)
    d1 = _down_block(y2, d0, sc, sh, _w_taps(enc_l1_down_w), enc_l1_down_b,
                     nimg=4)

    m16 = b * 16 * 16
    w_eo = jnp.pad(jnp.transpose(enc_out_w[:, :, 0, 0]),
                   ((0, 0), (0, C - emb_dim))).astype(_BF16)
    b_eo = jnp.pad(enc_out_b, (0, C - emb_dim)).reshape(1, C).astype(_F32)
    w_di = jnp.pad(jnp.transpose(dec_in_w[:, :, 0, 0]),
                   ((0, C - emb_dim), (0, 0))).astype(_BF16)
    b_di = dec_in_b.reshape(1, C).astype(_F32)
    e_p = jnp.pad(codebook.astype(_F32), ((0, 0), (0, C - emb_dim)))
    e2 = jnp.sum(e_p * e_p, axis=-1).reshape(1, num_emb).astype(_F32)

    tm, steps = 4096, m16 // 4096
    row_spec = pl.BlockSpec((tm, C), lambda i: (i, 0))
    idx, cnts, hd = _pcall(
        _bridge_body, (steps,),
        [row_spec, _fix_spec((C, C)), _fix_spec((1, C)),
         _fix_spec((num_emb, C)), _fix_spec((1, num_emb)),
         _fix_spec((C, C)), _fix_spec((1, C))],
        (pl.BlockSpec((tm, 1), lambda i: (i, 0)),
         pl.BlockSpec((1, 1, num_emb), lambda i: (i, 0, 0)),
         row_spec),
        (jax.ShapeDtypeStruct((m16, 1), jnp.int32),
         jax.ShapeDtypeStruct((steps, 1, num_emb), _F32),
         jax.ShapeDtypeStruct((m16, C), _BF16)),
        (d1.reshape(m16, C), w_eo, b_eo, e_p, e2, w_di, b_di))

    counts = jnp.sum(cnts, axis=(0, 1))
    p = counts + 1e-6
    p = p / jnp.sum(p)
    entropy = -jnp.sum(p * jnp.log(p))
    loss = -entropy

    h0 = hd.reshape(b, 16, 16, C)

    y1, s1, q1 = _conv_block(h0, _w_taps(dec_l0_rb0_conv1_w),
                             dec_l0_rb0_conv1_b, nimg=16)
    sc, sh = _bn_scale_shift(jnp.sum(s1, axis=(0, 1)), jnp.sum(q1, axis=(0, 1)),
                             m16, dec_l0_rb0_bn1_g, dec_l0_rb0_bn1_b)
    y2, s2, q2 = _conv_block(y1, _w_taps(dec_l0_rb0_conv2_w),
                             dec_l0_rb0_conv2_b, nimg=16, aff=(sc, sh))
    sc, sh = _bn_scale_shift(jnp.sum(s2, axis=(0, 1)), jnp.sum(q2, axis=(0, 1)),
                             m16, dec_l0_rb0_bn2_g, dec_l0_rb0_bn2_b)
    bu0 = jnp.tile(dec_l0_up_b, 4).reshape(1, 4 * C).astype(_F32)
    u0 = _convt_block(y2, h0, sc, sh, _w_convt(dec_l0_up_w), bu0,
                      nimg=8)

    y1, s1, q1 = _conv_block(u0, _w_taps(dec_l1_rb0_conv1_w),
                             dec_l1_rb0_conv1_b, nimg=4)
    sc, sh = _bn_scale_shift(jnp.sum(s1, axis=(0, 1)), jnp.sum(q1, axis=(0, 1)),
                             m32, dec_l1_rb0_bn1_g, dec_l1_rb0_bn1_b)
    y2, s2, q2 = _conv_block(y1, _w_taps(dec_l1_rb0_conv2_w),
                             dec_l1_rb0_conv2_b, nimg=4, aff=(sc, sh))
    sc, sh = _bn_scale_shift(jnp.sum(s2, axis=(0, 1)), jnp.sum(q2, axis=(0, 1)),
                             m32, dec_l1_rb0_bn2_g, dec_l1_rb0_bn2_b)
    bu1 = jnp.tile(dec_l1_up_b, 4).reshape(1, 4 * C).astype(_F32)
    cout = dec_out_w.shape[0]
    cout8 = 8
    w_do = jnp.pad(jnp.transpose(dec_out_w[:, :, 0, 0]),
                   ((0, 0), (0, cout8 - cout))).astype(_BF16)
    b_do = jnp.pad(dec_out_b, (0, cout8 - cout)).reshape(1, cout8).astype(_F32)
    p00, p01, p10, p11 = _convt_out_block(
        y2, u0, sc, sh, _w_convt(dec_l1_up_w), bu1, w_do, b_do, nimg=2)

    t = jnp.stack([jnp.stack([p00, p01], axis=3),
                   jnp.stack([p10, p11], axis=3)], axis=2)
    recon = t.reshape(b, 64, 64, cout8)[..., :cout]
    recon = jnp.transpose(recon, (0, 3, 1, 2))

    return recon, loss, idx.reshape(b, 16, 16)


# speedup vs baseline: 16.3944x; 1.1237x over previous
import functools

import jax
import jax.numpy as jnp
from jax import lax
from jax.experimental import pallas as pl
from jax.experimental.pallas import tpu as pltpu

C = 128
TM = 256
VMEM_LIMIT = 32 * 1024 * 1024
_F32 = jnp.float32
_BF16 = jnp.bfloat16


def _w_taps(w):
    _, _, kh, kw = w.shape
    wt = jnp.transpose(w, (2, 3, 1, 0))
    return wt.reshape(kh * kw * w.shape[1], w.shape[0]).astype(_BF16)


def _w_convt(w):
    sel = ((3, 1), (2, 0))
    zero = jnp.zeros_like(w[:, :, 0, 0])
    taps = []
    for r in range(3):
        for c in range(3):
            blocks = []
            for a in (0, 1):
                for b in (0, 1):
                    di, dj = r - a, c - b
                    if 0 <= di <= 1 and 0 <= dj <= 1:
                        blocks.append(w[:, :, sel[a][di], sel[b][dj]])
                    else:
                        blocks.append(zero)
            taps.append(jnp.concatenate(blocks, axis=1))
    return jnp.concatenate(taps, axis=0).astype(_BF16)


def _bn_scale_shift(s, q, count, gamma, beta, eps=1e-5):
    mean = s / count
    var = jnp.maximum(q / count - mean * mean, 0.0)
    scale = gamma * lax.rsqrt(var + eps)
    shift = beta - mean * scale
    return scale.reshape(1, C).astype(_F32), shift.reshape(1, C).astype(_F32)


def _halo(a):
    return jnp.pad(a, ((0, 0), (1, 1), (1, 1), (0, 0)))


def _slab_dma(xp, kh, kw, ho, wo, nimg, slab_ref, slab2_ref, sem):
    rows = nimg * ho * wo
    for i in range(kh):
        for j in range(kw):
            t = i * kw + j
            slab_ref[:, t * C:(t + 1) * C] = (
                xp[:, i:i + ho, j:j + wo, :].reshape(rows, C))
    cp = pltpu.make_async_copy(slab_ref, slab2_ref, sem)
    cp.start()
    cp.wait()


def _gemm_chunks(slab2_ref, w_ref, b_ref, rows):
    n = w_ref.shape[-1]
    tn = 256 if (n % 256 == 0 and n >= 256) else n
    for r in range(rows // TM):
        a_c = slab2_ref[r * TM:(r + 1) * TM, :]
        if tn == n:
            yc = jnp.dot(a_c, w_ref[...], preferred_element_type=_F32) + b_ref[...]
        else:
            yc = jnp.concatenate(
                [jnp.dot(a_c, w_ref[:, c * tn:(c + 1) * tn],
                         preferred_element_type=_F32)
                 for c in range(n // tn)], axis=1) + b_ref[...]
        yield r, yc


def _in_conv1_body(x_ref, wi_ref, bi_ref, w_ref, b_ref,
                   y0_ref, y1_ref, s_ref, q_ref, slab_ref, slab2_ref, sem,
                   *, nimg, h, w):
    rows = nimg * h * w
    cin = x_ref.shape[-1]
    y0 = jnp.dot(x_ref[...].reshape(rows, cin), wi_ref[...],
                 preferred_element_type=_F32) + bi_ref[...]
    y0 = y0.astype(_BF16)
    y0_ref[...] = y0.reshape(nimg, h // 2, 2, w // 2, 2 * C)
    _slab_dma(_halo(y0.reshape(nimg, h, w, C)), 3, 3, h, w, nimg,
              slab_ref, slab2_ref, sem)
    for r, yc in _gemm_chunks(slab2_ref, w_ref, b_ref, rows):
        s_ref[r:r + 1, :, :] = jnp.sum(yc, axis=0, keepdims=True)[None]
        q_ref[r:r + 1, :, :] = jnp.sum(yc * yc, axis=0, keepdims=True)[None]
        y1_ref[r * TM:(r + 1) * TM, :] = yc.astype(_BF16)


def _conv1_body(x_ref, w_ref, b_ref, o_ref, s_ref, q_ref,
                slab_ref, slab2_ref, sem, *, nimg, h, w):
    rows = nimg * h * w
    _slab_dma(_halo(x_ref[...]), 3, 3, h, w, nimg, slab_ref, slab2_ref, sem)
    for r, yc in _gemm_chunks(slab2_ref, w_ref, b_ref, rows):
        s_ref[r:r + 1, :, :] = jnp.sum(yc, axis=0, keepdims=True)[None]
        q_ref[r:r + 1, :, :] = jnp.sum(yc * yc, axis=0, keepdims=True)[None]
        o_ref[r * TM:(r + 1) * TM, :] = yc.astype(_BF16)


def _aff_conv2_body(x_ref, sc_ref, sh_ref, w_ref, b_ref,
                    o_ref, s_ref, q_ref, slab_ref, slab2_ref, sem,
                    *, nimg, h, w, phase_out=False):
    rows = nimg * h * w
    a = jnp.maximum(x_ref[...].astype(_F32) * sc_ref[...] + sh_ref[...], 0.0)
    a = a.astype(_BF16)
    _slab_dma(_halo(a), 3, 3, h, w, nimg, slab_ref, slab2_ref, sem)
    nrh = TM // w
    for r, yc in _gemm_chunks(slab2_ref, w_ref, b_ref, rows):
        s_ref[r:r + 1, :, :] = jnp.sum(yc, axis=0, keepdims=True)[None]
        q_ref[r:r + 1, :, :] = jnp.sum(yc * yc, axis=0, keepdims=True)[None]
        yb = yc.astype(_BF16)
        if phase_out:
            img, lh = (r * TM) // (h * w), ((r * TM) % (h * w)) // w
            o_ref[img, lh // 2:(lh + nrh) // 2, :, :, :] = (
                yb.reshape(nrh // 2, 2, w // 2, 2 * C))
        else:
            o_ref[r * TM:(r + 1) * TM, :] = yb


def _tail_down_body(x0_ref, x1_ref, r0_ref, r1_ref, sc_ref, sh_ref,
                    w_ref, b_ref, o_ref, slab_ref, slab2_ref, sem,
                    *, nimg, h, w):
    ho, wo = h // 2, w // 2
    rows = nimg * ho * wo
    app = {}
    for a, xr, rr in ((0, x0_ref, r0_ref), (1, x1_ref, r1_ref)):
        xe = xr[...].reshape(nimg, ho, wo, 2 * C).astype(_F32)
        re = rr[...].reshape(nimg, ho, wo, 2 * C).astype(_F32)
        for b in (0, 1):
            act = jnp.maximum(
                xe[..., b * C:(b + 1) * C] * sc_ref[...] + sh_ref[...]
                + re[..., b * C:(b + 1) * C], 0.0).astype(_BF16)
            app[(a, b)] = jnp.pad(act, ((0, 0), (a, 1 - a), (b, 1 - b), (0, 0)))
    for i in range(4):
        for j in range(4):
            t = i * 4 + j
            p = app[(1 - i % 2, 1 - j % 2)]
            slab_ref[:, t * C:(t + 1) * C] = (
                p[:, i // 2:i // 2 + ho, j // 2:j // 2 + wo, :].reshape(rows, C))
    cp = pltpu.make_async_copy(slab_ref, slab2_ref, sem)
    cp.start()
    cp.wait()
    for r, yc in _gemm_chunks(slab2_ref, w_ref, b_ref, rows):
        o_ref[r * TM:(r + 1) * TM, :] = jnp.maximum(yc, 0.0).astype(_BF16)


def _tail_convt_body(x_ref, r_ref, sc_ref, sh_ref, w_ref, b_ref, o_ref,
                     slab_ref, slab2_ref, sem, *, nimg, h, w):
    rows = nimg * h * w
    a = jnp.maximum(x_ref[...].astype(_F32) * sc_ref[...] + sh_ref[...]
                    + r_ref[...].astype(_F32), 0.0).astype(_BF16)
    _slab_dma(_halo(a), 3, 3, h, w, nimg, slab_ref, slab2_ref, sem)
    for r, yc in _gemm_chunks(slab2_ref, w_ref, b_ref, rows):
        o_ref[r * TM:(r + 1) * TM, :] = jnp.maximum(yc, 0.0).astype(_BF16)


def _tail_convt_out_body(x_ref, r_ref, sc_ref, sh_ref, w_ref, b_ref,
                         wo_ref, bo_ref, p0_ref, p1_ref, p2_ref, p3_ref,
                         slab_ref, slab2_ref, sem, *, nimg, h, w):
    rows = nimg * h * w
    a = jnp.maximum(x_ref[...].astype(_F32) * sc_ref[...] + sh_ref[...]
                    + r_ref[...].astype(_F32), 0.0).astype(_BF16)
    _slab_dma(_halo(a), 3, 3, h, w, nimg, slab_ref, slab2_ref, sem)
    outs = (p0_ref, p1_ref, p2_ref, p3_ref)
    for r, yc in _gemm_chunks(slab2_ref, w_ref, b_ref, rows):
        y4 = jnp.maximum(yc, 0.0).astype(_BF16)
        for p, o_ref in enumerate(outs):
            yp = jnp.dot(y4[:, p * C:(p + 1) * C], wo_ref[...],
                         preferred_element_type=_F32) + bo_ref[...]
            o_ref[r * TM:(r + 1) * TM, :] = jax.nn.sigmoid(yp)


def _bridge_body(y_ref, wo_ref, bo_ref, e_ref, e2_ref, wd_ref, bd_ref,
                 idx_ref, cnt_ref, h_ref):
    rows = y_ref.shape[0]
    kdim = e_ref.shape[0]
    cnt = jnp.zeros((1, kdim), _F32)
    for r in range(rows // TM):
        sl = slice(r * TM, (r + 1) * TM)
        z = jnp.dot(y_ref[sl, :], wo_ref[...],
                    preferred_element_type=_F32) + bo_ref[...]
        z2 = jnp.sum(z * z, axis=-1, keepdims=True)
        cross = lax.dot_general(z, e_ref[...], (((1,), (1,)), ((), ())),
                                preferred_element_type=_F32)
        d = z2 - 2.0 * cross + e2_ref[...]
        d_min = jnp.min(d, axis=-1, keepdims=True)
        ids = lax.broadcasted_iota(jnp.int32, d.shape, 1)
        idx = jnp.min(jnp.where(d <= d_min, ids, kdim), axis=-1, keepdims=True)
        idx_ref[sl, :] = idx
        cnt = cnt + jnp.sum((ids == idx).astype(_F32), axis=0, keepdims=True)
        hd = jnp.dot(z.astype(_BF16), wd_ref[...],
                     preferred_element_type=_F32) + bd_ref[...]
        h_ref[sl, :] = hd.astype(_BF16)
    cnt_ref[...] = cnt[None]


def _pcall(body, grid, in_specs, out_specs, out_shape, args, slab=None):
    scratch = []
    if slab is not None:
        scratch = [pltpu.VMEM(slab, _BF16), pltpu.VMEM(slab, _BF16),
                   pltpu.SemaphoreType.DMA]
    return pl.pallas_call(
        body,
        out_shape=out_shape,
        grid_spec=pltpu.PrefetchScalarGridSpec(
            num_scalar_prefetch=0, grid=grid,
            in_specs=in_specs, out_specs=out_specs,
            scratch_shapes=scratch),
        compiler_params=pltpu.CompilerParams(
            dimension_semantics=("parallel",),
            vmem_limit_bytes=VMEM_LIMIT),
    )(*args)


def _img_spec(nimg, h, w, ch):
    return pl.BlockSpec((nimg, h, w, ch), lambda i: (i, 0, 0, 0))


def _row_spec(rows, ch):
    return pl.BlockSpec((rows, ch), lambda i: (i, 0))


def _fix_spec(shape):
    nd = len(shape)
    return pl.BlockSpec(shape, lambda i: (0,) * nd)


def _stat_specs_shapes(b, nimg, h, w):
    nchunk = b * h * w // TM
    per = nimg * h * w // TM
    spec = pl.BlockSpec((per, 1, C), lambda i: (i, 0, 0))
    shape = jax.ShapeDtypeStruct((nchunk, 1, C), _F32)
    return (spec, spec), (shape, shape)


def _conv_block(x, w9, bias, *, nimg, aff=None, fuse_in=None,
                phase_out=False):
    b, h, w, _ = x.shape
    rows = nimg * h * w
    grid = (b // nimg,)
    o_shape = jax.ShapeDtypeStruct((b * h * w, C), _BF16)
    st_specs, st_shapes = _stat_specs_shapes(b, nimg, h, w)
    bias = bias.reshape(1, -1).astype(_F32)
    ph_spec = pl.BlockSpec((nimg, h // 2, 2, w // 2, 2 * C),
                           lambda i: (i, 0, 0, 0, 0))
    ph_shape = jax.ShapeDtypeStruct((b, h // 2, 2, w // 2, 2 * C), _BF16)
    if fuse_in is not None:
        wi, bi = fuse_in
        body = functools.partial(_in_conv1_body, nimg=nimg, h=h, w=w)
        in_specs = [_img_spec(nimg, h, w, x.shape[-1]), _fix_spec(wi.shape),
                    _fix_spec((1, C)), _fix_spec(w9.shape), _fix_spec((1, C))]
        out_specs = (ph_spec, _row_spec(rows, C)) + st_specs
        out_shape = (ph_shape, o_shape) + st_shapes
        args = (x, wi, bi.reshape(1, C).astype(_F32), w9, bias)
    elif aff is not None:
        sc, sh = aff
        body = functools.partial(_aff_conv2_body, nimg=nimg, h=h, w=w,
                                 phase_out=phase_out)
        in_specs = [_img_spec(nimg, h, w, C), _fix_spec((1, C)), _fix_spec((1, C)),
                    _fix_spec(w9.shape), _fix_spec((1, C))]
        out_specs = ((ph_spec if phase_out else _row_spec(rows, C)),) + st_specs
        out_shape = ((ph_shape if phase_out else o_shape),) + st_shapes
        args = (x, sc, sh, w9, bias)
    else:
        body = functools.partial(_conv1_body, nimg=nimg, h=h, w=w)
        in_specs = [_img_spec(nimg, h, w, C), _fix_spec(w9.shape), _fix_spec((1, C))]
        out_specs = (_row_spec(rows, C),) + st_specs
        out_shape = (o_shape,) + st_shapes
        args = (x, w9, bias)
    res = _pcall(body, grid, in_specs, out_specs, out_shape, args,
                 slab=(rows, 9 * C))
    if fuse_in is not None:
        return (res[0], res[1].reshape(b, h, w, C)) + tuple(res[2:])
    if not phase_out:
        res = (res[0].reshape(b, h, w, C),) + tuple(res[1:])
    return res


def _down_block(y2v, resv, sc, sh, wd, bd, *, nimg):
    b, ho, _, wo, _ = y2v.shape
    h, w = 2 * ho, 2 * wo
    rows = nimg * ho * wo
    grid = (b // nimg,)
    body = functools.partial(_tail_down_body, nimg=nimg, h=h, w=w)

    def _phase_spec(e):
        return pl.BlockSpec((nimg, ho, 1, wo, 2 * C),
                            lambda i, e=e: (i, 0, e, 0, 0))

    in_specs = [_phase_spec(0), _phase_spec(1), _phase_spec(0), _phase_spec(1),
                _fix_spec((1, C)), _fix_spec((1, C)),
                _fix_spec(wd.shape), _fix_spec((1, C))]
    out = _pcall(body, grid, in_specs, _row_spec(rows, C),
                 jax.ShapeDtypeStruct((b * ho * wo, C), _BF16),
                 (y2v, y2v, resv, resv, sc, sh, wd,
                  bd.reshape(1, C).astype(_F32)),
                 slab=(rows, 16 * C))
    return out.reshape(b, ho, wo, C)


def _convt_block(y2, res, sc, sh, wu, bu4, *, nimg):
    b, h, w, _ = y2.shape
    rows = nimg * h * w
    grid = (b // nimg,)
    body = functools.partial(_tail_convt_body, nimg=nimg, h=h, w=w)
    in_specs = [_img_spec(nimg, h, w, C), _img_spec(nimg, h, w, C),
                _fix_spec((1, C)), _fix_spec((1, C)),
                _fix_spec(wu.shape), _fix_spec((1, 4 * C))]
    y4 = _pcall(body, grid, in_specs, _row_spec(rows, 4 * C),
                jax.ShapeDtypeStruct((b * h * w, 4 * C), _BF16),
                (y2, res, sc, sh, wu, bu4), slab=(rows, 9 * C))
    y4 = y4.reshape(b, h, w, 2, 2, C)
    return jnp.transpose(y4, (0, 1, 3, 2, 4, 5)).reshape(b, 2 * h, 2 * w, C)


def _convt_out_block(y2, res, sc, sh, wu, bu4, wo, bo, *, nimg):
    b, h, w, _ = y2.shape
    rows = nimg * h * w
    grid = (b // nimg,)
    couts = wo.shape[-1]
    body = functools.partial(_tail_convt_out_body, nimg=nimg, h=h, w=w)
    in_specs = [_img_spec(nimg, h, w, C), _img_spec(nimg, h, w, C),
                _fix_spec((1, C)), _fix_spec((1, C)),
                _fix_spec(wu.shape), _fix_spec((1, 4 * C)),
                _fix_spec(wo.shape), _fix_spec((1, couts))]
    p_spec = _row_spec(rows, couts)
    p_shape = jax.ShapeDtypeStruct((b * h * w, couts), _F32)
    ps = _pcall(body, grid, in_specs, (p_spec,) * 4, (p_shape,) * 4,
                (y2, res, sc, sh, wu, bu4, wo, bo), slab=(rows, 9 * C))
    return tuple(p.reshape(b, h, w, couts) for p in ps)


def kernel(x, enc_in_w, enc_in_b,
           enc_l0_rb0_conv1_w, enc_l0_rb0_conv1_b, enc_l0_rb0_bn1_g, enc_l0_rb0_bn1_b,
           enc_l0_rb0_conv2_w, enc_l0_rb0_conv2_b, enc_l0_rb0_bn2_g, enc_l0_rb0_bn2_b,
           enc_l0_down_w, enc_l0_down_b,
           enc_l1_rb0_conv1_w, enc_l1_rb0_conv1_b, enc_l1_rb0_bn1_g, enc_l1_rb0_bn1_b,
           enc_l1_rb0_conv2_w, enc_l1_rb0_conv2_b, enc_l1_rb0_bn2_g, enc_l1_rb0_bn2_b,
           enc_l1_down_w, enc_l1_down_b,
           enc_out_w, enc_out_b,
           dec_in_w, dec_in_b,
           dec_l0_rb0_conv1_w, dec_l0_rb0_conv1_b, dec_l0_rb0_bn1_g, dec_l0_rb0_bn1_b,
           dec_l0_rb0_conv2_w, dec_l0_rb0_conv2_b, dec_l0_rb0_bn2_g, dec_l0_rb0_bn2_b,
           dec_l0_up_w, dec_l0_up_b,
           dec_l1_rb0_conv1_w, dec_l1_rb0_conv1_b, dec_l1_rb0_bn1_g, dec_l1_rb0_bn1_b,
           dec_l1_rb0_conv2_w, dec_l1_rb0_conv2_b, dec_l1_rb0_bn2_g, dec_l1_rb0_bn2_b,
           dec_l1_up_w, dec_l1_up_b,
           dec_out_w, dec_out_b,
           codebook):
    b = x.shape[0]
    num_emb, emb_dim = codebook.shape

    x8 = jnp.transpose(x, (0, 2, 3, 1)).astype(_BF16)
    cin8 = 8
    x8 = jnp.pad(x8, ((0, 0), (0, 0), (0, 0), (0, cin8 - x8.shape[-1])))
    w_in = jnp.pad(jnp.transpose(enc_in_w[:, :, 0, 0]),
                   ((0, cin8 - enc_in_w.shape[1]), (0, 0))).astype(_BF16)

    m64 = b * 64 * 64
    y0, y1, s1, q1 = _conv_block(x8, _w_taps(enc_l0_rb0_conv1_w),
                                 enc_l0_rb0_conv1_b, nimg=1,
                                 fuse_in=(w_in, enc_in_b))
    sc, sh = _bn_scale_shift(jnp.sum(s1, axis=(0, 1)), jnp.sum(q1, axis=(0, 1)),
                             m64, enc_l0_rb0_bn1_g, enc_l0_rb0_bn1_b)
    y2, s2, q2 = _conv_block(y1, _w_taps(enc_l0_rb0_conv2_w),
                             enc_l0_rb0_conv2_b, nimg=1, aff=(sc, sh),
                             phase_out=True)
    sc, sh = _bn_scale_shift(jnp.sum(s2, axis=(0, 1)), jnp.sum(q2, axis=(0, 1)),
                             m64, enc_l0_rb0_bn2_g, enc_l0_rb0_bn2_b)
    d0 = _down_block(y2, y0, sc, sh, _w_taps(enc_l0_down_w), enc_l0_down_b,
                     nimg=1)

    m32 = b * 32 * 32
    y1, s1, q1 = _conv_block(d0, _w_taps(enc_l1_rb0_conv1_w),
                             enc_l1_rb0_conv1_b, nimg=4)
    sc, sh = _bn_scale_shift(jnp.sum(s1, axis=(0, 1)), jnp.sum(q1, axis=(0, 1)),
                             m32, enc_l1_rb0_bn1_g, enc_l1_rb0_bn1_b)
    y2, s2, q2 = _conv_block(y1, _w_taps(enc_l1_rb0_conv2_w),
                             enc_l1_rb0_conv2_b, nimg=4, aff=(sc, sh),
                             phase_out=True)
    sc, sh = _bn_scale_shift(jnp.sum(s2, axis=(0, 1)), jnp.sum(q2, axis=(0, 1)),
                             m32, enc_l1_rb0_bn2_g, enc_l1_rb0_bn2_b)
    d1 = _down_block(y2, d0.reshape(b, 16, 2, 16, 2 * C), sc, sh,
                     _w_taps(enc_l1_down_w), enc_l1_down_b,
                     nimg=4)

    m16 = b * 16 * 16
    w_eo = jnp.pad(jnp.transpose(enc_out_w[:, :, 0, 0]),
                   ((0, 0), (0, C - emb_dim))).astype(_BF16)
    b_eo = jnp.pad(enc_out_b, (0, C - emb_dim)).reshape(1, C).astype(_F32)
    w_di = jnp.pad(jnp.transpose(dec_in_w[:, :, 0, 0]),
                   ((0, C - emb_dim), (0, 0))).astype(_BF16)
    b_di = dec_in_b.reshape(1, C).astype(_F32)
    e_p = jnp.pad(codebook.astype(_F32), ((0, 0), (0, C - emb_dim)))
    e2 = jnp.sum(e_p * e_p, axis=-1).reshape(1, num_emb).astype(_F32)

    tm, steps = 4096, m16 // 4096
    row_spec = pl.BlockSpec((tm, C), lambda i: (i, 0))
    idx, cnts, hd = _pcall(
        _bridge_body, (steps,),
        [row_spec, _fix_spec((C, C)), _fix_spec((1, C)),
         _fix_spec((num_emb, C)), _fix_spec((1, num_emb)),
         _fix_spec((C, C)), _fix_spec((1, C))],
        (pl.BlockSpec((tm, 1), lambda i: (i, 0)),
         pl.BlockSpec((1, 1, num_emb), lambda i: (i, 0, 0)),
         row_spec),
        (jax.ShapeDtypeStruct((m16, 1), jnp.int32),
         jax.ShapeDtypeStruct((steps, 1, num_emb), _F32),
         jax.ShapeDtypeStruct((m16, C), _BF16)),
        (d1.reshape(m16, C), w_eo, b_eo, e_p, e2, w_di, b_di))

    counts = jnp.sum(cnts, axis=(0, 1))
    p = counts + 1e-6
    p = p / jnp.sum(p)
    entropy = -jnp.sum(p * jnp.log(p))
    loss = -entropy

    h0 = hd.reshape(b, 16, 16, C)

    y1, s1, q1 = _conv_block(h0, _w_taps(dec_l0_rb0_conv1_w),
                             dec_l0_rb0_conv1_b, nimg=16)
    sc, sh = _bn_scale_shift(jnp.sum(s1, axis=(0, 1)), jnp.sum(q1, axis=(0, 1)),
                             m16, dec_l0_rb0_bn1_g, dec_l0_rb0_bn1_b)
    y2, s2, q2 = _conv_block(y1, _w_taps(dec_l0_rb0_conv2_w),
                             dec_l0_rb0_conv2_b, nimg=16, aff=(sc, sh))
    sc, sh = _bn_scale_shift(jnp.sum(s2, axis=(0, 1)), jnp.sum(q2, axis=(0, 1)),
                             m16, dec_l0_rb0_bn2_g, dec_l0_rb0_bn2_b)
    bu0 = jnp.tile(dec_l0_up_b, 4).reshape(1, 4 * C).astype(_F32)
    u0 = _convt_block(y2, h0, sc, sh, _w_convt(dec_l0_up_w), bu0,
                      nimg=8)

    y1, s1, q1 = _conv_block(u0, _w_taps(dec_l1_rb0_conv1_w),
                             dec_l1_rb0_conv1_b, nimg=4)
    sc, sh = _bn_scale_shift(jnp.sum(s1, axis=(0, 1)), jnp.sum(q1, axis=(0, 1)),
                             m32, dec_l1_rb0_bn1_g, dec_l1_rb0_bn1_b)
    y2, s2, q2 = _conv_block(y1, _w_taps(dec_l1_rb0_conv2_w),
                             dec_l1_rb0_conv2_b, nimg=4, aff=(sc, sh))
    sc, sh = _bn_scale_shift(jnp.sum(s2, axis=(0, 1)), jnp.sum(q2, axis=(0, 1)),
                             m32, dec_l1_rb0_bn2_g, dec_l1_rb0_bn2_b)
    bu1 = jnp.tile(dec_l1_up_b, 4).reshape(1, 4 * C).astype(_F32)
    cout = dec_out_w.shape[0]
    cout8 = 8
    w_do = jnp.pad(jnp.transpose(dec_out_w[:, :, 0, 0]),
                   ((0, 0), (0, cout8 - cout))).astype(_BF16)
    b_do = jnp.pad(dec_out_b, (0, cout8 - cout)).reshape(1, cout8).astype(_F32)
    p00, p01, p10, p11 = _convt_out_block(
        y2, u0, sc, sh, _w_convt(dec_l1_up_w), bu1, w_do, b_do, nimg=2)

    t = jnp.stack([jnp.stack([p00, p01], axis=3),
                   jnp.stack([p10, p11], axis=3)], axis=2)
    recon = t.reshape(b, 64, 64, cout8)[..., :cout]
    recon = jnp.transpose(recon, (0, 3, 1, 2))

    return recon, loss, idx.reshape(b, 16, 16)


# speedup vs baseline: 16.6732x; 1.0170x over previous
import functools

import jax
import jax.numpy as jnp
from jax import lax
from jax.experimental import pallas as pl
from jax.experimental.pallas import tpu as pltpu

C = 128
TM = 256
VMEM_LIMIT = 32 * 1024 * 1024
_F32 = jnp.float32
_BF16 = jnp.bfloat16


def _w_taps(w):
    _, _, kh, kw = w.shape
    wt = jnp.transpose(w, (2, 3, 1, 0))
    return wt.reshape(kh * kw * w.shape[1], w.shape[0]).astype(_BF16)


def _w_convt(w):
    sel = ((3, 1), (2, 0))
    zero = jnp.zeros_like(w[:, :, 0, 0])
    taps = []
    for r in range(3):
        for c in range(3):
            blocks = []
            for a in (0, 1):
                for b in (0, 1):
                    di, dj = r - a, c - b
                    if 0 <= di <= 1 and 0 <= dj <= 1:
                        blocks.append(w[:, :, sel[a][di], sel[b][dj]])
                    else:
                        blocks.append(zero)
            taps.append(jnp.concatenate(blocks, axis=1))
    return jnp.concatenate(taps, axis=0).astype(_BF16)


def _bn_scale_shift(s, q, count, gamma, beta, eps=1e-5):
    mean = s / count
    var = jnp.maximum(q / count - mean * mean, 0.0)
    scale = gamma * lax.rsqrt(var + eps)
    shift = beta - mean * scale
    return scale.reshape(1, C).astype(_F32), shift.reshape(1, C).astype(_F32)


def _halo(a):
    return jnp.pad(a, ((0, 0), (1, 1), (1, 1), (0, 0)))


def _slab_dma(xp, kh, kw, ho, wo, nimg, slab_ref, slab2_ref, sem):
    rows = nimg * ho * wo
    for i in range(kh):
        for j in range(kw):
            t = i * kw + j
            slab_ref[:, t * C:(t + 1) * C] = (
                xp[:, i:i + ho, j:j + wo, :].reshape(rows, C))
    cp = pltpu.make_async_copy(slab_ref, slab2_ref, sem)
    cp.start()
    cp.wait()


def _gemm_chunks(slab2_ref, w_ref, b_ref, rows):
    n = w_ref.shape[-1]
    tn = 256 if (n % 256 == 0 and n >= 256) else n
    for r in range(rows // TM):
        a_c = slab2_ref[r * TM:(r + 1) * TM, :]
        if tn == n:
            yc = jnp.dot(a_c, w_ref[...], preferred_element_type=_F32) + b_ref[...]
        else:
            yc = jnp.concatenate(
                [jnp.dot(a_c, w_ref[:, c * tn:(c + 1) * tn],
                         preferred_element_type=_F32)
                 for c in range(n // tn)], axis=1) + b_ref[...]
        yield r, yc


def _in_conv1_body(x_ref, wi_ref, bi_ref, w_ref, b_ref,
                   y0_ref, y1_ref, s_ref, q_ref, slab_ref, slab2_ref, sem,
                   *, nimg, h, w):
    rows = nimg * h * w
    cin = x_ref.shape[-1]
    y0 = jnp.dot(x_ref[...].reshape(rows, cin), wi_ref[...],
                 preferred_element_type=_F32) + bi_ref[...]
    y0 = y0.astype(_BF16)
    y0_ref[...] = y0.reshape(nimg, h // 2, 2, w // 2, 2 * C)
    _slab_dma(_halo(y0.reshape(nimg, h, w, C)), 3, 3, h, w, nimg,
              slab_ref, slab2_ref, sem)
    for r, yc in _gemm_chunks(slab2_ref, w_ref, b_ref, rows):
        s_ref[r:r + 1, :, :] = jnp.sum(yc, axis=0, keepdims=True)[None]
        q_ref[r:r + 1, :, :] = jnp.sum(yc * yc, axis=0, keepdims=True)[None]
        y1_ref[r * TM:(r + 1) * TM, :] = yc.astype(_BF16)


def _conv1_body(x_ref, w_ref, b_ref, o_ref, s_ref, q_ref,
                slab_ref, slab2_ref, sem, *, nimg, h, w):
    rows = nimg * h * w
    _slab_dma(_halo(x_ref[...]), 3, 3, h, w, nimg, slab_ref, slab2_ref, sem)
    for r, yc in _gemm_chunks(slab2_ref, w_ref, b_ref, rows):
        s_ref[r:r + 1, :, :] = jnp.sum(yc, axis=0, keepdims=True)[None]
        q_ref[r:r + 1, :, :] = jnp.sum(yc * yc, axis=0, keepdims=True)[None]
        o_ref[r * TM:(r + 1) * TM, :] = yc.astype(_BF16)


def _aff_conv2_body(x_ref, sc_ref, sh_ref, w_ref, b_ref,
                    o_ref, s_ref, q_ref, slab_ref, slab2_ref, sem,
                    *, nimg, h, w, phase_out=False):
    rows = nimg * h * w
    a = jnp.maximum(x_ref[...].astype(_F32) * sc_ref[...] + sh_ref[...], 0.0)
    a = a.astype(_BF16)
    _slab_dma(_halo(a), 3, 3, h, w, nimg, slab_ref, slab2_ref, sem)
    nrh = TM // w
    for r, yc in _gemm_chunks(slab2_ref, w_ref, b_ref, rows):
        s_ref[r:r + 1, :, :] = jnp.sum(yc, axis=0, keepdims=True)[None]
        q_ref[r:r + 1, :, :] = jnp.sum(yc * yc, axis=0, keepdims=True)[None]
        yb = yc.astype(_BF16)
        if phase_out:
            img, lh = (r * TM) // (h * w), ((r * TM) % (h * w)) // w
            o_ref[img, lh // 2:(lh + nrh) // 2, :, :, :] = (
                yb.reshape(nrh // 2, 2, w // 2, 2 * C))
        else:
            o_ref[r * TM:(r + 1) * TM, :] = yb


def _tail_down_body(x0_ref, x1_ref, r0_ref, r1_ref, sc_ref, sh_ref,
                    w_ref, b_ref, o_ref, slab_ref, slab2_ref, sem,
                    *, nimg, h, w):
    ho, wo = h // 2, w // 2
    rows = nimg * ho * wo
    app = {}
    for a, xr, rr in ((0, x0_ref, r0_ref), (1, x1_ref, r1_ref)):
        xe = xr[...].reshape(nimg, ho, wo, 2 * C).astype(_F32)
        re = rr[...].reshape(nimg, ho, wo, 2 * C).astype(_F32)
        for b in (0, 1):
            act = jnp.maximum(
                xe[..., b * C:(b + 1) * C] * sc_ref[...] + sh_ref[...]
                + re[..., b * C:(b + 1) * C], 0.0).astype(_BF16)
            app[(a, b)] = jnp.pad(act, ((0, 0), (a, 1 - a), (b, 1 - b), (0, 0)))
    for i in range(4):
        for j in range(4):
            t = i * 4 + j
            p = app[(1 - i % 2, 1 - j % 2)]
            slab_ref[:, t * C:(t + 1) * C] = (
                p[:, i // 2:i // 2 + ho, j // 2:j // 2 + wo, :].reshape(rows, C))
    cp = pltpu.make_async_copy(slab_ref, slab2_ref, sem)
    cp.start()
    cp.wait()
    for r, yc in _gemm_chunks(slab2_ref, w_ref, b_ref, rows):
        o_ref[r * TM:(r + 1) * TM, :] = jnp.maximum(yc, 0.0).astype(_BF16)


def _tail_convt_body(x_ref, r_ref, sc_ref, sh_ref, w_ref, b_ref, o_ref,
                     slab_ref, slab2_ref, sem, *, nimg, h, w):
    rows = nimg * h * w
    a = jnp.maximum(x_ref[...].astype(_F32) * sc_ref[...] + sh_ref[...]
                    + r_ref[...].astype(_F32), 0.0).astype(_BF16)
    _slab_dma(_halo(a), 3, 3, h, w, nimg, slab_ref, slab2_ref, sem)
    for r, yc in _gemm_chunks(slab2_ref, w_ref, b_ref, rows):
        y4 = jnp.maximum(yc, 0.0).astype(_BF16)
        t = y4.reshape(h, w, 2, 2, C)
        o_ref[r] = jnp.transpose(t, (0, 2, 1, 3, 4)).reshape(2 * h, 2 * w, C)


def _tail_convt_out_body(x_ref, r_ref, sc_ref, sh_ref, w_ref, b_ref,
                         wo_ref, bo_ref, p0_ref, p1_ref, p2_ref, p3_ref,
                         slab_ref, slab2_ref, sem, *, nimg, h, w):
    rows = nimg * h * w
    a = jnp.maximum(x_ref[...].astype(_F32) * sc_ref[...] + sh_ref[...]
                    + r_ref[...].astype(_F32), 0.0).astype(_BF16)
    _slab_dma(_halo(a), 3, 3, h, w, nimg, slab_ref, slab2_ref, sem)
    outs = (p0_ref, p1_ref, p2_ref, p3_ref)
    for r, yc in _gemm_chunks(slab2_ref, w_ref, b_ref, rows):
        y4 = jnp.maximum(yc, 0.0).astype(_BF16)
        for p, o_ref in enumerate(outs):
            yp = jnp.dot(y4[:, p * C:(p + 1) * C], wo_ref[...],
                         preferred_element_type=_F32) + bo_ref[...]
            o_ref[r * TM:(r + 1) * TM, :] = jax.nn.sigmoid(yp)


def _bridge_body(y_ref, wo_ref, bo_ref, e_ref, e2_ref, wd_ref, bd_ref,
                 idx_ref, cnt_ref, h_ref):
    rows = y_ref.shape[0]
    kdim = e_ref.shape[0]
    cnt = jnp.zeros((1, kdim), _F32)
    for r in range(rows // TM):
        sl = slice(r * TM, (r + 1) * TM)
        z = jnp.dot(y_ref[sl, :], wo_ref[...],
                    preferred_element_type=_F32) + bo_ref[...]
        z2 = jnp.sum(z * z, axis=-1, keepdims=True)
        cross = lax.dot_general(z, e_ref[...], (((1,), (1,)), ((), ())),
                                preferred_element_type=_F32)
        d = z2 - 2.0 * cross + e2_ref[...]
        d_min = jnp.min(d, axis=-1, keepdims=True)
        ids = lax.broadcasted_iota(jnp.int32, d.shape, 1)
        idx = jnp.min(jnp.where(d <= d_min, ids, kdim), axis=-1, keepdims=True)
        idx_ref[sl, :] = idx
        cnt = cnt + jnp.sum((ids == idx).astype(_F32), axis=0, keepdims=True)
        hd = jnp.dot(z.astype(_BF16), wd_ref[...],
                     preferred_element_type=_F32) + bd_ref[...]
        h_ref[sl, :] = hd.astype(_BF16)
    cnt_ref[...] = cnt[None]


def _pcall(body, grid, in_specs, out_specs, out_shape, args, slab=None):
    scratch = []
    if slab is not None:
        scratch = [pltpu.VMEM(slab, _BF16), pltpu.VMEM(slab, _BF16),
                   pltpu.SemaphoreType.DMA]
    return pl.pallas_call(
        body,
        out_shape=out_shape,
        grid_spec=pltpu.PrefetchScalarGridSpec(
            num_scalar_prefetch=0, grid=grid,
            in_specs=in_specs, out_specs=out_specs,
            scratch_shapes=scratch),
        compiler_params=pltpu.CompilerParams(
            dimension_semantics=("parallel",),
            vmem_limit_bytes=VMEM_LIMIT),
    )(*args)


def _img_spec(nimg, h, w, ch):
    return pl.BlockSpec((nimg, h, w, ch), lambda i: (i, 0, 0, 0))


def _row_spec(rows, ch):
    return pl.BlockSpec((rows, ch), lambda i: (i, 0))


def _fix_spec(shape):
    nd = len(shape)
    return pl.BlockSpec(shape, lambda i: (0,) * nd)


def _stat_specs_shapes(b, nimg, h, w):
    nchunk = b * h * w // TM
    per = nimg * h * w // TM
    spec = pl.BlockSpec((per, 1, C), lambda i: (i, 0, 0))
    shape = jax.ShapeDtypeStruct((nchunk, 1, C), _F32)
    return (spec, spec), (shape, shape)


def _conv_block(x, w9, bias, *, nimg, aff=None, fuse_in=None,
                phase_out=False):
    b, h, w, _ = x.shape
    rows = nimg * h * w
    grid = (b // nimg,)
    o_shape = jax.ShapeDtypeStruct((b * h * w, C), _BF16)
    st_specs, st_shapes = _stat_specs_shapes(b, nimg, h, w)
    bias = bias.reshape(1, -1).astype(_F32)
    ph_spec = pl.BlockSpec((nimg, h // 2, 2, w // 2, 2 * C),
                           lambda i: (i, 0, 0, 0, 0))
    ph_shape = jax.ShapeDtypeStruct((b, h // 2, 2, w // 2, 2 * C), _BF16)
    if fuse_in is not None:
        wi, bi = fuse_in
        body = functools.partial(_in_conv1_body, nimg=nimg, h=h, w=w)
        in_specs = [_img_spec(nimg, h, w, x.shape[-1]), _fix_spec(wi.shape),
                    _fix_spec((1, C)), _fix_spec(w9.shape), _fix_spec((1, C))]
        out_specs = (ph_spec, _row_spec(rows, C)) + st_specs
        out_shape = (ph_shape, o_shape) + st_shapes
        args = (x, wi, bi.reshape(1, C).astype(_F32), w9, bias)
    elif aff is not None:
        sc, sh = aff
        body = functools.partial(_aff_conv2_body, nimg=nimg, h=h, w=w,
                                 phase_out=phase_out)
        in_specs = [_img_spec(nimg, h, w, C), _fix_spec((1, C)), _fix_spec((1, C)),
                    _fix_spec(w9.shape), _fix_spec((1, C))]
        out_specs = ((ph_spec if phase_out else _row_spec(rows, C)),) + st_specs
        out_shape = ((ph_shape if phase_out else o_shape),) + st_shapes
        args = (x, sc, sh, w9, bias)
    else:
        body = functools.partial(_conv1_body, nimg=nimg, h=h, w=w)
        in_specs = [_img_spec(nimg, h, w, C), _fix_spec(w9.shape), _fix_spec((1, C))]
        out_specs = (_row_spec(rows, C),) + st_specs
        out_shape = (o_shape,) + st_shapes
        args = (x, w9, bias)
    res = _pcall(body, grid, in_specs, out_specs, out_shape, args,
                 slab=(rows, 9 * C))
    if fuse_in is not None:
        return (res[0], res[1].reshape(b, h, w, C)) + tuple(res[2:])
    if not phase_out:
        res = (res[0].reshape(b, h, w, C),) + tuple(res[1:])
    return res


def _down_block(y2v, resv, sc, sh, wd, bd, *, nimg):
    b, ho, _, wo, _ = y2v.shape
    h, w = 2 * ho, 2 * wo
    rows = nimg * ho * wo
    grid = (b // nimg,)
    body = functools.partial(_tail_down_body, nimg=nimg, h=h, w=w)

    def _phase_spec(e):
        return pl.BlockSpec((nimg, ho, 1, wo, 2 * C),
                            lambda i, e=e: (i, 0, e, 0, 0))

    in_specs = [_phase_spec(0), _phase_spec(1), _phase_spec(0), _phase_spec(1),
                _fix_spec((1, C)), _fix_spec((1, C)),
                _fix_spec(wd.shape), _fix_spec((1, C))]
    out = _pcall(body, grid, in_specs, _row_spec(rows, C),
                 jax.ShapeDtypeStruct((b * ho * wo, C), _BF16),
                 (y2v, y2v, resv, resv, sc, sh, wd,
                  bd.reshape(1, C).astype(_F32)),
                 slab=(rows, 16 * C))
    return out.reshape(b, ho, wo, C)


def _convt_block(y2, res, sc, sh, wu, bu4, *, nimg):
    b, h, w, _ = y2.shape
    rows = nimg * h * w
    grid = (b // nimg,)
    assert h * w == TM
    body = functools.partial(_tail_convt_body, nimg=nimg, h=h, w=w)
    in_specs = [_img_spec(nimg, h, w, C), _img_spec(nimg, h, w, C),
                _fix_spec((1, C)), _fix_spec((1, C)),
                _fix_spec(wu.shape), _fix_spec((1, 4 * C))]
    return _pcall(body, grid, in_specs, _img_spec(nimg, 2 * h, 2 * w, C),
                  jax.ShapeDtypeStruct((b, 2 * h, 2 * w, C), _BF16),
                  (y2, res, sc, sh, wu, bu4), slab=(rows, 9 * C))


def _convt_out_block(y2, res, sc, sh, wu, bu4, wo, bo, *, nimg):
    b, h, w, _ = y2.shape
    rows = nimg * h * w
    grid = (b // nimg,)
    couts = wo.shape[-1]
    body = functools.partial(_tail_convt_out_body, nimg=nimg, h=h, w=w)
    in_specs = [_img_spec(nimg, h, w, C), _img_spec(nimg, h, w, C),
                _fix_spec((1, C)), _fix_spec((1, C)),
                _fix_spec(wu.shape), _fix_spec((1, 4 * C)),
                _fix_spec(wo.shape), _fix_spec((1, couts))]
    p_spec = _row_spec(rows, couts)
    p_shape = jax.ShapeDtypeStruct((b * h * w, couts), _F32)
    ps = _pcall(body, grid, in_specs, (p_spec,) * 4, (p_shape,) * 4,
                (y2, res, sc, sh, wu, bu4, wo, bo), slab=(rows, 9 * C))
    return tuple(p.reshape(b, h, w, couts) for p in ps)


def kernel(x, enc_in_w, enc_in_b,
           enc_l0_rb0_conv1_w, enc_l0_rb0_conv1_b, enc_l0_rb0_bn1_g, enc_l0_rb0_bn1_b,
           enc_l0_rb0_conv2_w, enc_l0_rb0_conv2_b, enc_l0_rb0_bn2_g, enc_l0_rb0_bn2_b,
           enc_l0_down_w, enc_l0_down_b,
           enc_l1_rb0_conv1_w, enc_l1_rb0_conv1_b, enc_l1_rb0_bn1_g, enc_l1_rb0_bn1_b,
           enc_l1_rb0_conv2_w, enc_l1_rb0_conv2_b, enc_l1_rb0_bn2_g, enc_l1_rb0_bn2_b,
           enc_l1_down_w, enc_l1_down_b,
           enc_out_w, enc_out_b,
           dec_in_w, dec_in_b,
           dec_l0_rb0_conv1_w, dec_l0_rb0_conv1_b, dec_l0_rb0_bn1_g, dec_l0_rb0_bn1_b,
           dec_l0_rb0_conv2_w, dec_l0_rb0_conv2_b, dec_l0_rb0_bn2_g, dec_l0_rb0_bn2_b,
           dec_l0_up_w, dec_l0_up_b,
           dec_l1_rb0_conv1_w, dec_l1_rb0_conv1_b, dec_l1_rb0_bn1_g, dec_l1_rb0_bn1_b,
           dec_l1_rb0_conv2_w, dec_l1_rb0_conv2_b, dec_l1_rb0_bn2_g, dec_l1_rb0_bn2_b,
           dec_l1_up_w, dec_l1_up_b,
           dec_out_w, dec_out_b,
           codebook):
    b = x.shape[0]
    num_emb, emb_dim = codebook.shape

    x8 = jnp.transpose(x, (0, 2, 3, 1)).astype(_BF16)
    cin8 = 8
    x8 = jnp.pad(x8, ((0, 0), (0, 0), (0, 0), (0, cin8 - x8.shape[-1])))
    w_in = jnp.pad(jnp.transpose(enc_in_w[:, :, 0, 0]),
                   ((0, cin8 - enc_in_w.shape[1]), (0, 0))).astype(_BF16)

    m64 = b * 64 * 64
    y0, y1, s1, q1 = _conv_block(x8, _w_taps(enc_l0_rb0_conv1_w),
                                 enc_l0_rb0_conv1_b, nimg=1,
                                 fuse_in=(w_in, enc_in_b))
    sc, sh = _bn_scale_shift(jnp.sum(s1, axis=(0, 1)), jnp.sum(q1, axis=(0, 1)),
                             m64, enc_l0_rb0_bn1_g, enc_l0_rb0_bn1_b)
    y2, s2, q2 = _conv_block(y1, _w_taps(enc_l0_rb0_conv2_w),
                             enc_l0_rb0_conv2_b, nimg=1, aff=(sc, sh),
                             phase_out=True)
    sc, sh = _bn_scale_shift(jnp.sum(s2, axis=(0, 1)), jnp.sum(q2, axis=(0, 1)),
                             m64, enc_l0_rb0_bn2_g, enc_l0_rb0_bn2_b)
    d0 = _down_block(y2, y0, sc, sh, _w_taps(enc_l0_down_w), enc_l0_down_b,
                     nimg=1)

    m32 = b * 32 * 32
    y1, s1, q1 = _conv_block(d0, _w_taps(enc_l1_rb0_conv1_w),
                             enc_l1_rb0_conv1_b, nimg=4)
    sc, sh = _bn_scale_shift(jnp.sum(s1, axis=(0, 1)), jnp.sum(q1, axis=(0, 1)),
                             m32, enc_l1_rb0_bn1_g, enc_l1_rb0_bn1_b)
    y2, s2, q2 = _conv_block(y1, _w_taps(enc_l1_rb0_conv2_w),
                             enc_l1_rb0_conv2_b, nimg=4, aff=(sc, sh),
                             phase_out=True)
    sc, sh = _bn_scale_shift(jnp.sum(s2, axis=(0, 1)), jnp.sum(q2, axis=(0, 1)),
                             m32, enc_l1_rb0_bn2_g, enc_l1_rb0_bn2_b)
    d1 = _down_block(y2, d0.reshape(b, 16, 2, 16, 2 * C), sc, sh,
                     _w_taps(enc_l1_down_w), enc_l1_down_b,
                     nimg=4)

    m16 = b * 16 * 16
    w_eo = jnp.pad(jnp.transpose(enc_out_w[:, :, 0, 0]),
                   ((0, 0), (0, C - emb_dim))).astype(_BF16)
    b_eo = jnp.pad(enc_out_b, (0, C - emb_dim)).reshape(1, C).astype(_F32)
    w_di = jnp.pad(jnp.transpose(dec_in_w[:, :, 0, 0]),
                   ((0, C - emb_dim), (0, 0))).astype(_BF16)
    b_di = dec_in_b.reshape(1, C).astype(_F32)
    e_p = jnp.pad(codebook.astype(_F32), ((0, 0), (0, C - emb_dim)))
    e2 = jnp.sum(e_p * e_p, axis=-1).reshape(1, num_emb).astype(_F32)

    tm, steps = 4096, m16 // 4096
    row_spec = pl.BlockSpec((tm, C), lambda i: (i, 0))
    idx, cnts, hd = _pcall(
        _bridge_body, (steps,),
        [row_spec, _fix_spec((C, C)), _fix_spec((1, C)),
         _fix_spec((num_emb, C)), _fix_spec((1, num_emb)),
         _fix_spec((C, C)), _fix_spec((1, C))],
        (pl.BlockSpec((tm, 1), lambda i: (i, 0)),
         pl.BlockSpec((1, 1, num_emb), lambda i: (i, 0, 0)),
         row_spec),
        (jax.ShapeDtypeStruct((m16, 1), jnp.int32),
         jax.ShapeDtypeStruct((steps, 1, num_emb), _F32),
         jax.ShapeDtypeStruct((m16, C), _BF16)),
        (d1.reshape(m16, C), w_eo, b_eo, e_p, e2, w_di, b_di))

    counts = jnp.sum(cnts, axis=(0, 1))
    p = counts + 1e-6
    p = p / jnp.sum(p)
    entropy = -jnp.sum(p * jnp.log(p))
    loss = -entropy

    h0 = hd.reshape(b, 16, 16, C)

    y1, s1, q1 = _conv_block(h0, _w_taps(dec_l0_rb0_conv1_w),
                             dec_l0_rb0_conv1_b, nimg=16)
    sc, sh = _bn_scale_shift(jnp.sum(s1, axis=(0, 1)), jnp.sum(q1, axis=(0, 1)),
                             m16, dec_l0_rb0_bn1_g, dec_l0_rb0_bn1_b)
    y2, s2, q2 = _conv_block(y1, _w_taps(dec_l0_rb0_conv2_w),
                             dec_l0_rb0_conv2_b, nimg=16, aff=(sc, sh))
    sc, sh = _bn_scale_shift(jnp.sum(s2, axis=(0, 1)), jnp.sum(q2, axis=(0, 1)),
                             m16, dec_l0_rb0_bn2_g, dec_l0_rb0_bn2_b)
    bu0 = jnp.tile(dec_l0_up_b, 4).reshape(1, 4 * C).astype(_F32)
    u0 = _convt_block(y2, h0, sc, sh, _w_convt(dec_l0_up_w), bu0,
                      nimg=8)

    y1, s1, q1 = _conv_block(u0, _w_taps(dec_l1_rb0_conv1_w),
                             dec_l1_rb0_conv1_b, nimg=4)
    sc, sh = _bn_scale_shift(jnp.sum(s1, axis=(0, 1)), jnp.sum(q1, axis=(0, 1)),
                             m32, dec_l1_rb0_bn1_g, dec_l1_rb0_bn1_b)
    y2, s2, q2 = _conv_block(y1, _w_taps(dec_l1_rb0_conv2_w),
                             dec_l1_rb0_conv2_b, nimg=4, aff=(sc, sh))
    sc, sh = _bn_scale_shift(jnp.sum(s2, axis=(0, 1)), jnp.sum(q2, axis=(0, 1)),
                             m32, dec_l1_rb0_bn2_g, dec_l1_rb0_bn2_b)
    bu1 = jnp.tile(dec_l1_up_b, 4).reshape(1, 4 * C).astype(_F32)
    cout = dec_out_w.shape[0]
    cout8 = 8
    w_do = jnp.pad(jnp.transpose(dec_out_w[:, :, 0, 0]),
                   ((0, 0), (0, cout8 - cout))).astype(_BF16)
    b_do = jnp.pad(dec_out_b, (0, cout8 - cout)).reshape(1, cout8).astype(_F32)
    p00, p01, p10, p11 = _convt_out_block(
        y2, u0, sc, sh, _w_convt(dec_l1_up_w), bu1, w_do, b_do, nimg=2)

    t = jnp.stack([jnp.stack([p00, p01], axis=3),
                   jnp.stack([p10, p11], axis=3)], axis=2)
    recon = t.reshape(b, 64, 64, cout8)[..., :cout]
    recon = jnp.transpose(recon, (0, 3, 1, 2))

    return recon, loss, idx.reshape(b, 16, 16)


# speedup vs baseline: 16.7274x; 1.0033x over previous
import functools

import jax
import jax.numpy as jnp
from jax import lax
from jax.experimental import pallas as pl
from jax.experimental.pallas import tpu as pltpu

C = 128
TM = 256
VMEM_LIMIT = 32 * 1024 * 1024
_F32 = jnp.float32
_BF16 = jnp.bfloat16


def _w_taps(w):
    _, _, kh, kw = w.shape
    wt = jnp.transpose(w, (2, 3, 1, 0))
    return wt.reshape(kh * kw * w.shape[1], w.shape[0]).astype(_BF16)


def _w_convt(w):
    sel = ((3, 1), (2, 0))
    zero = jnp.zeros_like(w[:, :, 0, 0])
    taps = []
    for r in range(3):
        for c in range(3):
            blocks = []
            for a in (0, 1):
                for b in (0, 1):
                    di, dj = r - a, c - b
                    if 0 <= di <= 1 and 0 <= dj <= 1:
                        blocks.append(w[:, :, sel[a][di], sel[b][dj]])
                    else:
                        blocks.append(zero)
            taps.append(jnp.concatenate(blocks, axis=1))
    return jnp.concatenate(taps, axis=0).astype(_BF16)


def _bn_scale_shift(s, q, count, gamma, beta, eps=1e-5):
    mean = s / count
    var = jnp.maximum(q / count - mean * mean, 0.0)
    scale = gamma * lax.rsqrt(var + eps)
    shift = beta - mean * scale
    return scale.reshape(1, C).astype(_F32), shift.reshape(1, C).astype(_F32)


def _halo(a):
    return jnp.pad(a, ((0, 0), (1, 1), (1, 1), (0, 0)))


def _slab_dma(xp, kh, kw, ho, wo, nimg, slab_ref, slab2_ref, sem):
    rows = nimg * ho * wo
    for i in range(kh):
        for j in range(kw):
            t = i * kw + j
            slab_ref[:, t * C:(t + 1) * C] = (
                xp[:, i:i + ho, j:j + wo, :].reshape(rows, C))
    cp = pltpu.make_async_copy(slab_ref, slab2_ref, sem)
    cp.start()
    cp.wait()


def _gemm_chunks(slab2_ref, w_ref, b_ref, rows):
    n = w_ref.shape[-1]
    tn = 256 if (n % 256 == 0 and n >= 256) else n
    for r in range(rows // TM):
        a_c = slab2_ref[r * TM:(r + 1) * TM, :]
        if tn == n:
            yc = jnp.dot(a_c, w_ref[...], preferred_element_type=_F32) + b_ref[...]
        else:
            yc = jnp.concatenate(
                [jnp.dot(a_c, w_ref[:, c * tn:(c + 1) * tn],
                         preferred_element_type=_F32)
                 for c in range(n // tn)], axis=1) + b_ref[...]
        yield r, yc


def _in_conv1_body(x_ref, wi_ref, bi_ref, w_ref, b_ref,
                   y0_ref, y1_ref, s_ref, q_ref, slab_ref, slab2_ref, sem,
                   *, nimg, h, w):
    rows = nimg * h * w
    cin = x_ref.shape[-1]
    y0 = jnp.dot(x_ref[...].reshape(rows, cin), wi_ref[...],
                 preferred_element_type=_F32) + bi_ref[...]
    y0 = y0.astype(_BF16)
    y0_ref[...] = y0.reshape(nimg, h // 2, 2, w // 2, 2 * C)
    _slab_dma(_halo(y0.reshape(nimg, h, w, C)), 3, 3, h, w, nimg,
              slab_ref, slab2_ref, sem)
    for r, yc in _gemm_chunks(slab2_ref, w_ref, b_ref, rows):
        s_ref[r:r + 1, :, :] = jnp.sum(yc, axis=0, keepdims=True)[None]
        q_ref[r:r + 1, :, :] = jnp.sum(yc * yc, axis=0, keepdims=True)[None]
        y1_ref[r * TM:(r + 1) * TM, :] = yc.astype(_BF16)


def _conv1_body(x_ref, w_ref, b_ref, o_ref, s_ref, q_ref,
                slab_ref, slab2_ref, sem, *, nimg, h, w):
    rows = nimg * h * w
    _slab_dma(_halo(x_ref[...]), 3, 3, h, w, nimg, slab_ref, slab2_ref, sem)
    for r, yc in _gemm_chunks(slab2_ref, w_ref, b_ref, rows):
        s_ref[r:r + 1, :, :] = jnp.sum(yc, axis=0, keepdims=True)[None]
        q_ref[r:r + 1, :, :] = jnp.sum(yc * yc, axis=0, keepdims=True)[None]
        o_ref[r * TM:(r + 1) * TM, :] = yc.astype(_BF16)


def _aff_conv2_body(x_ref, sc_ref, sh_ref, w_ref, b_ref,
                    o_ref, s_ref, q_ref, slab_ref, slab2_ref, sem,
                    *, nimg, h, w, phase_out=False):
    rows = nimg * h * w
    a = jnp.maximum(x_ref[...].astype(_F32) * sc_ref[...] + sh_ref[...], 0.0)
    a = a.astype(_BF16)
    _slab_dma(_halo(a), 3, 3, h, w, nimg, slab_ref, slab2_ref, sem)
    nrh = TM // w
    for r, yc in _gemm_chunks(slab2_ref, w_ref, b_ref, rows):
        s_ref[r:r + 1, :, :] = jnp.sum(yc, axis=0, keepdims=True)[None]
        q_ref[r:r + 1, :, :] = jnp.sum(yc * yc, axis=0, keepdims=True)[None]
        yb = yc.astype(_BF16)
        if phase_out:
            img, lh = (r * TM) // (h * w), ((r * TM) % (h * w)) // w
            o_ref[img, lh // 2:(lh + nrh) // 2, :, :, :] = (
                yb.reshape(nrh // 2, 2, w // 2, 2 * C))
        else:
            o_ref[r * TM:(r + 1) * TM, :] = yb


def _tail_down_body(x0_ref, x1_ref, r0_ref, r1_ref, sc_ref, sh_ref,
                    w_ref, b_ref, o_ref, slab_ref, slab2_ref, sem,
                    *, nimg, h, w):
    ho, wo = h // 2, w // 2
    rows = nimg * ho * wo
    app = {}
    for a, xr, rr in ((0, x0_ref, r0_ref), (1, x1_ref, r1_ref)):
        xe = xr[...].reshape(nimg, ho, wo, 2 * C).astype(_F32)
        re = rr[...].reshape(nimg, ho, wo, 2 * C).astype(_F32)
        for b in (0, 1):
            act = jnp.maximum(
                xe[..., b * C:(b + 1) * C] * sc_ref[...] + sh_ref[...]
                + re[..., b * C:(b + 1) * C], 0.0).astype(_BF16)
            app[(a, b)] = jnp.pad(act, ((0, 0), (a, 1 - a), (b, 1 - b), (0, 0)))
    for i in range(4):
        for j in range(4):
            t = i * 4 + j
            p = app[(1 - i % 2, 1 - j % 2)]
            slab_ref[:, t * C:(t + 1) * C] = (
                p[:, i // 2:i // 2 + ho, j // 2:j // 2 + wo, :].reshape(rows, C))
    cp = pltpu.make_async_copy(slab_ref, slab2_ref, sem)
    cp.start()
    cp.wait()
    for r, yc in _gemm_chunks(slab2_ref, w_ref, b_ref, rows):
        o_ref[r * TM:(r + 1) * TM, :] = jnp.maximum(yc, 0.0).astype(_BF16)


def _tail_convt_body(x_ref, r_ref, sc_ref, sh_ref, w_ref, b_ref, o_ref,
                     slab_ref, slab2_ref, sem, *, nimg, h, w):
    rows = nimg * h * w
    a = jnp.maximum(x_ref[...].astype(_F32) * sc_ref[...] + sh_ref[...]
                    + r_ref[...].astype(_F32), 0.0).astype(_BF16)
    _slab_dma(_halo(a), 3, 3, h, w, nimg, slab_ref, slab2_ref, sem)
    for r, yc in _gemm_chunks(slab2_ref, w_ref, b_ref, rows):
        y4 = jnp.maximum(yc, 0.0).astype(_BF16)
        t = y4.reshape(h, w, 2, 2, C)
        o_ref[r] = jnp.transpose(t, (0, 2, 1, 3, 4)).reshape(2 * h, 2 * w, C)


def _tail_convt_out_body(x_ref, r_ref, sc_ref, sh_ref, w_ref, b_ref,
                         wo_ref, bo_ref, p0_ref, p1_ref, p2_ref, p3_ref,
                         slab_ref, slab2_ref, sem, *, nimg, h, w):
    rows = nimg * h * w
    a = jnp.maximum(x_ref[...].astype(_F32) * sc_ref[...] + sh_ref[...]
                    + r_ref[...].astype(_F32), 0.0).astype(_BF16)
    _slab_dma(_halo(a), 3, 3, h, w, nimg, slab_ref, slab2_ref, sem)
    outs = (p0_ref, p1_ref, p2_ref, p3_ref)
    couts = wo_ref.shape[-1]
    nrh = TM // w
    for r, yc in _gemm_chunks(slab2_ref, w_ref, b_ref, rows):
        y4 = jnp.maximum(yc, 0.0).astype(_BF16)
        img, lh = (r * TM) // (h * w), ((r * TM) % (h * w)) // w
        for p, o_ref in enumerate(outs):
            yp = jnp.dot(y4[:, p * C:(p + 1) * C], wo_ref[...],
                         preferred_element_type=_F32) + bo_ref[...]
            yp = jax.nn.sigmoid(yp).reshape(nrh, w, couts)
            o_ref[img, :, lh:lh + nrh, :] = jnp.transpose(yp, (2, 0, 1))


def _bridge_body(y_ref, wo_ref, bo_ref, e_ref, e2_ref, wd_ref, bd_ref,
                 idx_ref, cnt_ref, h_ref):
    rows = y_ref.shape[0]
    kdim = e_ref.shape[0]
    cnt = jnp.zeros((1, kdim), _F32)
    for r in range(rows // TM):
        sl = slice(r * TM, (r + 1) * TM)
        z = jnp.dot(y_ref[sl, :], wo_ref[...],
                    preferred_element_type=_F32) + bo_ref[...]
        z2 = jnp.sum(z * z, axis=-1, keepdims=True)
        cross = lax.dot_general(z, e_ref[...], (((1,), (1,)), ((), ())),
                                preferred_element_type=_F32)
        d = z2 - 2.0 * cross + e2_ref[...]
        d_min = jnp.min(d, axis=-1, keepdims=True)
        ids = lax.broadcasted_iota(jnp.int32, d.shape, 1)
        idx = jnp.min(jnp.where(d <= d_min, ids, kdim), axis=-1, keepdims=True)
        idx_ref[sl, :] = idx
        cnt = cnt + jnp.sum((ids == idx).astype(_F32), axis=0, keepdims=True)
        hd = jnp.dot(z.astype(_BF16), wd_ref[...],
                     preferred_element_type=_F32) + bd_ref[...]
        h_ref[sl, :] = hd.astype(_BF16)
    cnt_ref[...] = cnt[None]


def _pcall(body, grid, in_specs, out_specs, out_shape, args, slab=None):
    scratch = []
    if slab is not None:
        scratch = [pltpu.VMEM(slab, _BF16), pltpu.VMEM(slab, _BF16),
                   pltpu.SemaphoreType.DMA]
    return pl.pallas_call(
        body,
        out_shape=out_shape,
        grid_spec=pltpu.PrefetchScalarGridSpec(
            num_scalar_prefetch=0, grid=grid,
            in_specs=in_specs, out_specs=out_specs,
            scratch_shapes=scratch),
        compiler_params=pltpu.CompilerParams(
            dimension_semantics=("parallel",),
            vmem_limit_bytes=VMEM_LIMIT),
    )(*args)


def _img_spec(nimg, h, w, ch):
    return pl.BlockSpec((nimg, h, w, ch), lambda i: (i, 0, 0, 0))


def _row_spec(rows, ch):
    return pl.BlockSpec((rows, ch), lambda i: (i, 0))


def _fix_spec(shape):
    nd = len(shape)
    return pl.BlockSpec(shape, lambda i: (0,) * nd)


def _stat_specs_shapes(b, nimg, h, w):
    nchunk = b * h * w // TM
    per = nimg * h * w // TM
    spec = pl.BlockSpec((per, 1, C), lambda i: (i, 0, 0))
    shape = jax.ShapeDtypeStruct((nchunk, 1, C), _F32)
    return (spec, spec), (shape, shape)


def _conv_block(x, w9, bias, *, nimg, aff=None, fuse_in=None,
                phase_out=False):
    b, h, w, _ = x.shape
    rows = nimg * h * w
    grid = (b // nimg,)
    o_shape = jax.ShapeDtypeStruct((b * h * w, C), _BF16)
    st_specs, st_shapes = _stat_specs_shapes(b, nimg, h, w)
    bias = bias.reshape(1, -1).astype(_F32)
    ph_spec = pl.BlockSpec((nimg, h // 2, 2, w // 2, 2 * C),
                           lambda i: (i, 0, 0, 0, 0))
    ph_shape = jax.ShapeDtypeStruct((b, h // 2, 2, w // 2, 2 * C), _BF16)
    if fuse_in is not None:
        wi, bi = fuse_in
        body = functools.partial(_in_conv1_body, nimg=nimg, h=h, w=w)
        in_specs = [_img_spec(nimg, h, w, x.shape[-1]), _fix_spec(wi.shape),
                    _fix_spec((1, C)), _fix_spec(w9.shape), _fix_spec((1, C))]
        out_specs = (ph_spec, _row_spec(rows, C)) + st_specs
        out_shape = (ph_shape, o_shape) + st_shapes
        args = (x, wi, bi.reshape(1, C).astype(_F32), w9, bias)
    elif aff is not None:
        sc, sh = aff
        body = functools.partial(_aff_conv2_body, nimg=nimg, h=h, w=w,
                                 phase_out=phase_out)
        in_specs = [_img_spec(nimg, h, w, C), _fix_spec((1, C)), _fix_spec((1, C)),
                    _fix_spec(w9.shape), _fix_spec((1, C))]
        out_specs = ((ph_spec if phase_out else _row_spec(rows, C)),) + st_specs
        out_shape = ((ph_shape if phase_out else o_shape),) + st_shapes
        args = (x, sc, sh, w9, bias)
    else:
        body = functools.partial(_conv1_body, nimg=nimg, h=h, w=w)
        in_specs = [_img_spec(nimg, h, w, C), _fix_spec(w9.shape), _fix_spec((1, C))]
        out_specs = (_row_spec(rows, C),) + st_specs
        out_shape = (o_shape,) + st_shapes
        args = (x, w9, bias)
    res = _pcall(body, grid, in_specs, out_specs, out_shape, args,
                 slab=(rows, 9 * C))
    if fuse_in is not None:
        return (res[0], res[1].reshape(b, h, w, C)) + tuple(res[2:])
    if not phase_out:
        res = (res[0].reshape(b, h, w, C),) + tuple(res[1:])
    return res


def _down_block(y2v, resv, sc, sh, wd, bd, *, nimg):
    b, ho, _, wo, _ = y2v.shape
    h, w = 2 * ho, 2 * wo
    rows = nimg * ho * wo
    grid = (b // nimg,)
    body = functools.partial(_tail_down_body, nimg=nimg, h=h, w=w)

    def _phase_spec(e):
        return pl.BlockSpec((nimg, ho, 1, wo, 2 * C),
                            lambda i, e=e: (i, 0, e, 0, 0))

    in_specs = [_phase_spec(0), _phase_spec(1), _phase_spec(0), _phase_spec(1),
                _fix_spec((1, C)), _fix_spec((1, C)),
                _fix_spec(wd.shape), _fix_spec((1, C))]
    out = _pcall(body, grid, in_specs, _row_spec(rows, C),
                 jax.ShapeDtypeStruct((b * ho * wo, C), _BF16),
                 (y2v, y2v, resv, resv, sc, sh, wd,
                  bd.reshape(1, C).astype(_F32)),
                 slab=(rows, 16 * C))
    return out.reshape(b, ho, wo, C)


def _convt_block(y2, res, sc, sh, wu, bu4, *, nimg):
    b, h, w, _ = y2.shape
    rows = nimg * h * w
    grid = (b // nimg,)
    assert h * w == TM
    body = functools.partial(_tail_convt_body, nimg=nimg, h=h, w=w)
    in_specs = [_img_spec(nimg, h, w, C), _img_spec(nimg, h, w, C),
                _fix_spec((1, C)), _fix_spec((1, C)),
                _fix_spec(wu.shape), _fix_spec((1, 4 * C))]
    return _pcall(body, grid, in_specs, _img_spec(nimg, 2 * h, 2 * w, C),
                  jax.ShapeDtypeStruct((b, 2 * h, 2 * w, C), _BF16),
                  (y2, res, sc, sh, wu, bu4), slab=(rows, 9 * C))


def _convt_out_block(y2, res, sc, sh, wu, bu4, wo, bo, *, nimg):
    b, h, w, _ = y2.shape
    rows = nimg * h * w
    grid = (b // nimg,)
    couts = wo.shape[-1]
    body = functools.partial(_tail_convt_out_body, nimg=nimg, h=h, w=w)
    in_specs = [_img_spec(nimg, h, w, C), _img_spec(nimg, h, w, C),
                _fix_spec((1, C)), _fix_spec((1, C)),
                _fix_spec(wu.shape), _fix_spec((1, 4 * C)),
                _fix_spec(wo.shape), _fix_spec((1, couts))]
    p_spec = pl.BlockSpec((nimg, couts, h, w), lambda i: (i, 0, 0, 0))
    p_shape = jax.ShapeDtypeStruct((b, couts, h, w), _F32)
    return _pcall(body, grid, in_specs, (p_spec,) * 4, (p_shape,) * 4,
                  (y2, res, sc, sh, wu, bu4, wo, bo), slab=(rows, 9 * C))


def kernel(x, enc_in_w, enc_in_b,
           enc_l0_rb0_conv1_w, enc_l0_rb0_conv1_b, enc_l0_rb0_bn1_g, enc_l0_rb0_bn1_b,
           enc_l0_rb0_conv2_w, enc_l0_rb0_conv2_b, enc_l0_rb0_bn2_g, enc_l0_rb0_bn2_b,
           enc_l0_down_w, enc_l0_down_b,
           enc_l1_rb0_conv1_w, enc_l1_rb0_conv1_b, enc_l1_rb0_bn1_g, enc_l1_rb0_bn1_b,
           enc_l1_rb0_conv2_w, enc_l1_rb0_conv2_b, enc_l1_rb0_bn2_g, enc_l1_rb0_bn2_b,
           enc_l1_down_w, enc_l1_down_b,
           enc_out_w, enc_out_b,
           dec_in_w, dec_in_b,
           dec_l0_rb0_conv1_w, dec_l0_rb0_conv1_b, dec_l0_rb0_bn1_g, dec_l0_rb0_bn1_b,
           dec_l0_rb0_conv2_w, dec_l0_rb0_conv2_b, dec_l0_rb0_bn2_g, dec_l0_rb0_bn2_b,
           dec_l0_up_w, dec_l0_up_b,
           dec_l1_rb0_conv1_w, dec_l1_rb0_conv1_b, dec_l1_rb0_bn1_g, dec_l1_rb0_bn1_b,
           dec_l1_rb0_conv2_w, dec_l1_rb0_conv2_b, dec_l1_rb0_bn2_g, dec_l1_rb0_bn2_b,
           dec_l1_up_w, dec_l1_up_b,
           dec_out_w, dec_out_b,
           codebook):
    b = x.shape[0]
    num_emb, emb_dim = codebook.shape

    x8 = jnp.transpose(x, (0, 2, 3, 1)).astype(_BF16)
    cin8 = 8
    x8 = jnp.pad(x8, ((0, 0), (0, 0), (0, 0), (0, cin8 - x8.shape[-1])))
    w_in = jnp.pad(jnp.transpose(enc_in_w[:, :, 0, 0]),
                   ((0, cin8 - enc_in_w.shape[1]), (0, 0))).astype(_BF16)

    m64 = b * 64 * 64
    y0, y1, s1, q1 = _conv_block(x8, _w_taps(enc_l0_rb0_conv1_w),
                                 enc_l0_rb0_conv1_b, nimg=1,
                                 fuse_in=(w_in, enc_in_b))
    sc, sh = _bn_scale_shift(jnp.sum(s1, axis=(0, 1)), jnp.sum(q1, axis=(0, 1)),
                             m64, enc_l0_rb0_bn1_g, enc_l0_rb0_bn1_b)
    y2, s2, q2 = _conv_block(y1, _w_taps(enc_l0_rb0_conv2_w),
                             enc_l0_rb0_conv2_b, nimg=1, aff=(sc, sh),
                             phase_out=True)
    sc, sh = _bn_scale_shift(jnp.sum(s2, axis=(0, 1)), jnp.sum(q2, axis=(0, 1)),
                             m64, enc_l0_rb0_bn2_g, enc_l0_rb0_bn2_b)
    d0 = _down_block(y2, y0, sc, sh, _w_taps(enc_l0_down_w), enc_l0_down_b,
                     nimg=1)

    m32 = b * 32 * 32
    y1, s1, q1 = _conv_block(d0, _w_taps(enc_l1_rb0_conv1_w),
                             enc_l1_rb0_conv1_b, nimg=4)
    sc, sh = _bn_scale_shift(jnp.sum(s1, axis=(0, 1)), jnp.sum(q1, axis=(0, 1)),
                             m32, enc_l1_rb0_bn1_g, enc_l1_rb0_bn1_b)
    y2, s2, q2 = _conv_block(y1, _w_taps(enc_l1_rb0_conv2_w),
                             enc_l1_rb0_conv2_b, nimg=4, aff=(sc, sh),
                             phase_out=True)
    sc, sh = _bn_scale_shift(jnp.sum(s2, axis=(0, 1)), jnp.sum(q2, axis=(0, 1)),
                             m32, enc_l1_rb0_bn2_g, enc_l1_rb0_bn2_b)
    d1 = _down_block(y2, d0.reshape(b, 16, 2, 16, 2 * C), sc, sh,
                     _w_taps(enc_l1_down_w), enc_l1_down_b,
                     nimg=4)

    m16 = b * 16 * 16
    w_eo = jnp.pad(jnp.transpose(enc_out_w[:, :, 0, 0]),
                   ((0, 0), (0, C - emb_dim))).astype(_BF16)
    b_eo = jnp.pad(enc_out_b, (0, C - emb_dim)).reshape(1, C).astype(_F32)
    w_di = jnp.pad(jnp.transpose(dec_in_w[:, :, 0, 0]),
                   ((0, C - emb_dim), (0, 0))).astype(_BF16)
    b_di = dec_in_b.reshape(1, C).astype(_F32)
    e_p = jnp.pad(codebook.astype(_F32), ((0, 0), (0, C - emb_dim)))
    e2 = jnp.sum(e_p * e_p, axis=-1).reshape(1, num_emb).astype(_F32)

    tm, steps = 4096, m16 // 4096
    row_spec = pl.BlockSpec((tm, C), lambda i: (i, 0))
    idx, cnts, hd = _pcall(
        _bridge_body, (steps,),
        [row_spec, _fix_spec((C, C)), _fix_spec((1, C)),
         _fix_spec((num_emb, C)), _fix_spec((1, num_emb)),
         _fix_spec((C, C)), _fix_spec((1, C))],
        (pl.BlockSpec((tm, 1), lambda i: (i, 0)),
         pl.BlockSpec((1, 1, num_emb), lambda i: (i, 0, 0)),
         row_spec),
        (jax.ShapeDtypeStruct((m16, 1), jnp.int32),
         jax.ShapeDtypeStruct((steps, 1, num_emb), _F32),
         jax.ShapeDtypeStruct((m16, C), _BF16)),
        (d1.reshape(m16, C), w_eo, b_eo, e_p, e2, w_di, b_di))

    counts = jnp.sum(cnts, axis=(0, 1))
    p = counts + 1e-6
    p = p / jnp.sum(p)
    entropy = -jnp.sum(p * jnp.log(p))
    loss = -entropy

    h0 = hd.reshape(b, 16, 16, C)

    y1, s1, q1 = _conv_block(h0, _w_taps(dec_l0_rb0_conv1_w),
                             dec_l0_rb0_conv1_b, nimg=16)
    sc, sh = _bn_scale_shift(jnp.sum(s1, axis=(0, 1)), jnp.sum(q1, axis=(0, 1)),
                             m16, dec_l0_rb0_bn1_g, dec_l0_rb0_bn1_b)
    y2, s2, q2 = _conv_block(y1, _w_taps(dec_l0_rb0_conv2_w),
                             dec_l0_rb0_conv2_b, nimg=16, aff=(sc, sh))
    sc, sh = _bn_scale_shift(jnp.sum(s2, axis=(0, 1)), jnp.sum(q2, axis=(0, 1)),
                             m16, dec_l0_rb0_bn2_g, dec_l0_rb0_bn2_b)
    bu0 = jnp.tile(dec_l0_up_b, 4).reshape(1, 4 * C).astype(_F32)
    u0 = _convt_block(y2, h0, sc, sh, _w_convt(dec_l0_up_w), bu0,
                      nimg=8)

    y1, s1, q1 = _conv_block(u0, _w_taps(dec_l1_rb0_conv1_w),
                             dec_l1_rb0_conv1_b, nimg=4)
    sc, sh = _bn_scale_shift(jnp.sum(s1, axis=(0, 1)), jnp.sum(q1, axis=(0, 1)),
                             m32, dec_l1_rb0_bn1_g, dec_l1_rb0_bn1_b)
    y2, s2, q2 = _conv_block(y1, _w_taps(dec_l1_rb0_conv2_w),
                             dec_l1_rb0_conv2_b, nimg=4, aff=(sc, sh))
    sc, sh = _bn_scale_shift(jnp.sum(s2, axis=(0, 1)), jnp.sum(q2, axis=(0, 1)),
                             m32, dec_l1_rb0_bn2_g, dec_l1_rb0_bn2_b)
    bu1 = jnp.tile(dec_l1_up_b, 4).reshape(1, 4 * C).astype(_F32)
    cout = dec_out_w.shape[0]
    cout8 = 8
    w_do = jnp.pad(jnp.transpose(dec_out_w[:, :, 0, 0]),
                   ((0, 0), (0, cout8 - cout))).astype(_BF16)
    b_do = jnp.pad(dec_out_b, (0, cout8 - cout)).reshape(1, cout8).astype(_F32)
    p00, p01, p10, p11 = _convt_out_block(
        y2, u0, sc, sh, _w_convt(dec_l1_up_w), bu1, w_do, b_do, nimg=2)

    t = jnp.stack([jnp.stack([p00, p01], axis=4),
                   jnp.stack([p10, p11], axis=4)], axis=3)
    recon = t.reshape(b, cout8, 64, 64)[:, :cout]

    return recon, loss, idx.reshape(b, 16, 16)
